# Initial kernel scaffold; baseline (speedup 1.0000x reference)
import jax
import jax.numpy as jnp
from jax.experimental import pallas as pl


def kernel(x, edge_index, root_index, W, b, att):
    raise NotImplementedError("write your pallas kernel here")


# speedup vs baseline: 7.4562x; 7.4562x over previous
import functools
import jax
import jax.numpy as jnp
from jax import lax
from jax.experimental import pallas as pl
from jax.experimental.pallas import tpu as pltpu
from jax.experimental.pallas import tpu_sc as plsc

N = 10000
E = 320000
D = 128
TEMP = 0.1
ROOTS = 1024
N_HOPS = 2

NC = 2
NS = 16
NT = NC * NS
EPT = E // NT
CH = 128
NCH = 80
EPTP = NCH * CH
NP = 10240
NPS = NP // NS
G8 = 8

_NEG = -1e30


def _seg_prefix(op, vals, keys, lane):
    out = vals
    for sh in (1, 2, 4, 8):
        idx = jnp.maximum(lane - sh, 0)
        vs = out.at[idx].get(mode="promise_in_bounds")
        ks = keys.at[idx].get(mode="promise_in_bounds")
        take = (ks == keys) & (lane >= sh)
        out = jnp.where(take, op(out, vs), out)
    return out


def _run_end(keys, lane):
    nidx = jnp.minimum(lane + 1, 15)
    knx = keys.at[nidx].get(mode="promise_in_bounds")
    return (knx != keys) | (lane == 15)


def _wid():
    return lax.axis_index("s") * NC + lax.axis_index("c")


def _sc_mesh():
    return plsc.VectorSubcoreMesh(core_axis_name="c", subcore_axis_name="s")


def _mm_body(f_ref, w_ref, b_ref, at_ref, h_ref, a_ref):
    h = jnp.dot(f_ref[...], w_ref[...], preferred_element_type=jnp.float32)
    h = h + b_ref[0:1, :]
    h_ref[...] = h
    a_ref[...] = jnp.dot(h, at_ref[...], preferred_element_type=jnp.float32)


def _mm_res_body(x_ref, p0_ref, p1_ref, w_ref, b_ref, at_ref,
                 f_ref, h_ref, a_ref):
    f = x_ref[...] + p0_ref[...] + p1_ref[...]
    f_ref[...] = f
    h = jnp.dot(f, w_ref[...], preferred_element_type=jnp.float32)
    h = h + b_ref[0:1, :]
    h_ref[...] = h
    a_ref[...] = jnp.dot(h, at_ref[...], preferred_element_type=jnp.float32)


_ROWS = 1000
_GRID = N // _ROWS


def _tc_project(f, W, b8, att2):
    return pl.pallas_call(
        _mm_body,
        grid=(_GRID,),
        in_specs=[
            pl.BlockSpec((_ROWS, D), lambda i: (i, 0)),
            pl.BlockSpec((D, D), lambda i: (0, 0)),
            pl.BlockSpec((8, D), lambda i: (0, 0)),
            pl.BlockSpec((D, D), lambda i: (0, 0)),
        ],
        out_specs=[
            pl.BlockSpec((_ROWS, D), lambda i: (i, 0)),
            pl.BlockSpec((_ROWS, D), lambda i: (i, 0)),
        ],
        out_shape=[
            jax.ShapeDtypeStruct((N, D), jnp.float32),
            jax.ShapeDtypeStruct((N, D), jnp.float32),
        ],
    )(f, W, b8, att2)


def _tc_project_res(x, p0, p1, W, b8, att2):
    return pl.pallas_call(
        _mm_res_body,
        grid=(_GRID,),
        in_specs=[
            pl.BlockSpec((_ROWS, D), lambda i: (i, 0)),
            pl.BlockSpec((_ROWS, D), lambda i: (i, 0)),
            pl.BlockSpec((_ROWS, D), lambda i: (i, 0)),
            pl.BlockSpec((D, D), lambda i: (0, 0)),
            pl.BlockSpec((8, D), lambda i: (0, 0)),
            pl.BlockSpec((D, D), lambda i: (0, 0)),
        ],
        out_specs=[
            pl.BlockSpec((_ROWS, D), lambda i: (i, 0)),
            pl.BlockSpec((_ROWS, D), lambda i: (i, 0)),
            pl.BlockSpec((_ROWS, D), lambda i: (i, 0)),
        ],
        out_shape=[
            jax.ShapeDtypeStruct((N, D), jnp.float32),
            jax.ShapeDtypeStruct((N, D), jnp.float32),
            jax.ShapeDtypeStruct((N, D), jnp.float32),
        ],
    )(x, p0, p1, W, b8, att2)


def _zm_body(ad_hbm, as_hbm, srcp, dstp, gp, z_out, m_parts,
             ad_t, as_t, src_t, dst_t, g_t, z_t, m_loc, mout, cmb, shm):
    c = lax.axis_index("c")
    s = lax.axis_index("s")
    wid = _wid()
    pltpu.sync_copy(ad_hbm, ad_t)
    pltpu.sync_copy(as_hbm, as_t)
    pltpu.sync_copy(srcp.at[wid], src_t)
    pltpu.sync_copy(dstp.at[wid], dst_t)
    pltpu.sync_copy(gp.at[wid], g_t)

    def init(i, _):
        m_loc[pl.ds(i * 16, 16)] = jnp.full((16,), _NEG, jnp.float32)
        return 0

    lax.fori_loop(0, NP // 16, init, 0)

    lane = lax.broadcasted_iota(jnp.int32, (16,), 0)

    def grp(r, _):
        for k in range(G8):
            sl = pl.ds(k * 16, 16)
            sv = src_t[r, sl]
            dv = dst_t[r, sl]
            gv = g_t[r, sl]
            av = plsc.load_gather(ad_t, [dv])
            bv = plsc.load_gather(as_t, [sv])
            t = av + bv
            lr = jnp.where(t >= 0.0, t, t * 0.2)
            z = (lr + gv) * (1.0 / TEMP)
            z_t[r, sl] = z
            dk, zk = plsc.sort_key_val(dv, z)
            zc = _seg_prefix(jnp.maximum, zk, dk, lane)
            end = _run_end(dk, lane)
            mo = plsc.load_gather(m_loc, [dk])
            plsc.store_scatter(m_loc, [dk], jnp.maximum(mo, zc), mask=end)
        return 0

    lax.fori_loop(0, NCH, grp, 0)
    pltpu.sync_copy(z_t, z_out.at[wid])

    pltpu.sync_copy(m_loc, shm.at[s])
    plsc.subcore_barrier()
    pltpu.sync_copy(shm.at[:, pl.ds(s * NPS, NPS)], cmb)

    def red(i, _):
        sl = pl.ds(i * 16, 16)
        acc = cmb[0, sl]
        for t in range(1, NS):
            acc = jnp.maximum(acc, cmb[t, sl])
        mout[sl] = acc
        return 0

    lax.fori_loop(0, NPS // 16, red, 0)
    pltpu.sync_copy(mout, m_parts.at[c, pl.ds(s * NPS, NPS)])


def _sc_zm(ad, asrc, srcp, dstp, gp):
    return pl.kernel(
        _zm_body,
        out_type=[
            jax.ShapeDtypeStruct((NT, NCH, CH), jnp.float32),
            jax.ShapeDtypeStruct((NC, NP), jnp.float32),
        ],
        mesh=_sc_mesh(),
        compiler_params=pltpu.CompilerParams(needs_layout_passes=False),
        scratch_types=[
            pltpu.VMEM((NP,), jnp.float32),
            pltpu.VMEM((NP,), jnp.float32),
            pltpu.VMEM((NCH, CH), jnp.int32),
            pltpu.VMEM((NCH, CH), jnp.int32),
            pltpu.VMEM((NCH, CH), jnp.float32),
            pltpu.VMEM((NCH, CH), jnp.float32),
            pltpu.VMEM((NP,), jnp.float32),
            pltpu.VMEM((NPS,), jnp.float32),
            pltpu.VMEM((NS, NPS), jnp.float32),
            pltpu.VMEM_SHARED((NS, NP), jnp.float32),
        ],
    )(ad, asrc, srcp, dstp, gp)


def _es_body(m_parts, z_in, dstp, e_out, s_parts,
             m_a, m_b, z_t, dst_t, e_t, s_loc, sout, cmb, shm):
    c = lax.axis_index("c")
    s = lax.axis_index("s")
    wid = _wid()
    pltpu.sync_copy(m_parts.at[0], m_a)
    pltpu.sync_copy(m_parts.at[1], m_b)
    pltpu.sync_copy(z_in.at[wid], z_t)
    pltpu.sync_copy(dstp.at[wid], dst_t)

    def init(i, _):
        sl = pl.ds(i * 16, 16)
        m_a[sl] = jnp.maximum(m_a[sl], m_b[sl])
        s_loc[sl] = jnp.zeros((16,), jnp.float32)
        return 0

    lax.fori_loop(0, NP // 16, init, 0)

    lane = lax.broadcasted_iota(jnp.int32, (16,), 0)

    def grp(r, _):
        for k in range(G8):
            sl = pl.ds(k * 16, 16)
            zv = z_t[r, sl]
            dv = dst_t[r, sl]
            mv = plsc.load_gather(m_a, [dv])
            ev = jnp.exp(zv - mv)
            e_t[r, sl] = ev
            dk, ek = plsc.sort_key_val(dv, ev)
            ec = _seg_prefix(jnp.add, ek, dk, lane)
            end = _run_end(dk, lane)
            so = plsc.load_gather(s_loc, [dk])
            plsc.store_scatter(s_loc, [dk], so + ec, mask=end)
        return 0

    lax.fori_loop(0, NCH, grp, 0)
    pltpu.sync_copy(e_t, e_out.at[wid])

    pltpu.sync_copy(s_loc, shm.at[s])
    plsc.subcore_barrier()
    pltpu.sync_copy(shm.at[:, pl.ds(s * NPS, NPS)], cmb)

    def red(i, _):
        sl = pl.ds(i * 16, 16)
        acc = cmb[0, sl]
        for t in range(1, NS):
            acc = acc + cmb[t, sl]
        sout[sl] = acc
        return 0

    lax.fori_loop(0, NPS // 16, red, 0)
    pltpu.sync_copy(sout, s_parts.at[c, pl.ds(s * NPS, NPS)])


def _sc_es(m_parts, z_in, dstp):
    return pl.kernel(
        _es_body,
        out_type=[
            jax.ShapeDtypeStruct((NT, NCH, CH), jnp.float32),
            jax.ShapeDtypeStruct((NC, NP), jnp.float32),
        ],
        mesh=_sc_mesh(),
        compiler_params=pltpu.CompilerParams(needs_layout_passes=False),
        scratch_types=[
            pltpu.VMEM((NP,), jnp.float32),
            pltpu.VMEM((NP,), jnp.float32),
            pltpu.VMEM((NCH, CH), jnp.float32),
            pltpu.VMEM((NCH, CH), jnp.int32),
            pltpu.VMEM((NCH, CH), jnp.float32),
            pltpu.VMEM((NP,), jnp.float32),
            pltpu.VMEM((NPS,), jnp.float32),
            pltpu.VMEM((NS, NPS), jnp.float32),
            pltpu.VMEM_SHARED((NS, NP), jnp.float32),
        ],
    )(m_parts, z_in, dstp)


def _agg_body(s_parts, e_in, srcp, dstp, h_hbm, out_parts,
              s_a, s_tmp, e_row, src_row, dst_row, alpha, rows, out_sh, sem):
    c = lax.axis_index("c")
    s = lax.axis_index("s")
    wid = _wid()
    pltpu.sync_copy(s_parts.at[0], s_a)

    for q in range(NP // NPS):
        pltpu.sync_copy(s_parts.at[1, pl.ds(q * NPS, NPS)], s_tmp)

        def init(i, _):
            sl = pl.ds(i * 16, 16)
            s_a[pl.ds(q * NPS + i * 16, 16)] = (
                s_a[pl.ds(q * NPS + i * 16, 16)] + s_tmp[sl])
            return 0

        lax.fori_loop(0, NPS // 16, init, 0)

    def zero_rows(r, _):
        for dcol in range(D // 16):
            rows[r, pl.ds(dcol * 16, 16)] = jnp.zeros((16,), jnp.float32)
        return 0

    lax.fori_loop(0, CH, zero_rows, 0)
    for q in range(NPS // CH):
        pltpu.sync_copy(rows, out_sh.at[pl.ds(s * NPS + q * CH, CH)])
    plsc.subcore_barrier()

    lane = lax.broadcasted_iota(jnp.int32, (16,), 0)

    def chunk(r, _):
        pltpu.sync_copy(e_in.at[wid, r], e_row)
        pltpu.sync_copy(srcp.at[wid, r], src_row)
        pltpu.sync_copy(dstp.at[wid, r], dst_row)
        for k in range(G8):
            sl = pl.ds(k * 16, 16)
            ev = e_row[sl]
            dv = dst_row[sl]
            sv = plsc.load_gather(s_a, [dv])
            av = ev / (sv + 1e-16)
            pos = r * CH + k * 16 + lane
            av = jnp.where(pos < EPT, av, 0.0)
            alpha[sl] = av
        pltpu.async_copy(h_hbm.at[src_row], rows, sem).wait()

        def scale(i, _):
            asp = plsc.load_gather(alpha, [jnp.full((16,), i, jnp.int32)])
            for dcol in range(D // 16):
                dsl = pl.ds(dcol * 16, 16)
                rows[i, dsl] = rows[i, dsl] * asp
            return 0

        lax.fori_loop(0, CH, scale, 0)
        pltpu.sync_copy(rows, out_sh.at[dst_row], add=True)
        return 0

    lax.fori_loop(0, NCH, chunk, 0)
    plsc.subcore_barrier()
    for q in range(NPS // CH):
        off = s * NPS + q * CH
        pltpu.sync_copy(out_sh.at[pl.ds(off, CH)],
                        out_parts.at[c, pl.ds(off, CH)])


def _sc_agg(s_parts, e_in, srcp, dstp, h):
    return pl.kernel(
        _agg_body,
        out_type=jax.ShapeDtypeStruct((NC, NP, D), jnp.float32),
        mesh=_sc_mesh(),
        compiler_params=pltpu.CompilerParams(needs_layout_passes=False),
        scratch_types=[
            pltpu.VMEM((NP,), jnp.float32),
            pltpu.VMEM((NPS,), jnp.float32),
            pltpu.VMEM((CH,), jnp.float32),
            pltpu.VMEM((CH,), jnp.int32),
            pltpu.VMEM((CH,), jnp.int32),
            pltpu.VMEM((CH,), jnp.float32),
            pltpu.VMEM((CH, D), jnp.float32),
            pltpu.VMEM_SHARED((NP, D), jnp.float32),
            pltpu.SemaphoreType.DMA,
        ],
    )(s_parts, e_in, srcp, dstp, h)


_RPT = ROOTS // NT


def _root_body(p0, p1, f1, roots, out, ridx, r0, r1, r2, o, sem):
    wid = _wid()
    sl = pl.ds(wid * _RPT, _RPT)
    pltpu.sync_copy(roots.at[sl], ridx)
    pltpu.async_copy(p0.at[ridx], r0, sem).wait()
    pltpu.async_copy(p1.at[ridx], r1, sem).wait()
    pltpu.async_copy(f1.at[ridx], r2, sem).wait()

    def add(i, _):
        for dcol in range(D // 16):
            dsl = pl.ds(dcol * 16, 16)
            o[i, dsl] = r0[i, dsl] + r1[i, dsl] + r2[i, dsl]
        return 0

    lax.fori_loop(0, _RPT, add, 0)
    pltpu.sync_copy(o, out.at[sl])


def _sc_root(p0, p1, f1, roots):
    return pl.kernel(
        _root_body,
        out_type=jax.ShapeDtypeStruct((ROOTS, D), jnp.float32),
        mesh=_sc_mesh(),
        compiler_params=pltpu.CompilerParams(needs_layout_passes=False),
        scratch_types=[
            pltpu.VMEM((_RPT,), jnp.int32),
            pltpu.VMEM((_RPT, D), jnp.float32),
            pltpu.VMEM((_RPT, D), jnp.float32),
            pltpu.VMEM((_RPT, D), jnp.float32),
            pltpu.VMEM((_RPT, D), jnp.float32),
            pltpu.SemaphoreType.DMA,
        ],
    )(p0, p1, f1, roots)


def _to_tiles(a, pad_val):
    a2 = a.reshape(NT, EPT)
    pad = jnp.full((NT, EPTP - EPT), pad_val, a2.dtype)
    return jnp.concatenate([a2, pad], axis=1).reshape(NT, NCH, CH)


def kernel(x, edge_index, root_index, W, b, att):
    key = jax.random.key(42)
    gp_hops = []
    for i in range(N_HOPS):
        u = jax.random.uniform(jax.random.fold_in(key, i), (E,),
                               minval=1e-6, maxval=1.0 - 1e-6)
        gp_hops.append(_to_tiles(-jnp.log(-jnp.log(u)), _NEG))

    srcp = _to_tiles(edge_index[0], 0)
    dstp = _to_tiles(edge_index[1], 0)
    b8 = jnp.broadcast_to(b.reshape(1, D), (8, D))
    att2 = jnp.zeros((D, D), jnp.float32)
    att2 = att2.at[:, 0].set(att[:D]).at[:, 1].set(att[D:])

    def hop(f, h, a, gp):
        ad = jnp.concatenate([a[:, 0], jnp.zeros((NP - N,), jnp.float32)])
        asrc = jnp.concatenate([a[:, 1], jnp.zeros((NP - N,), jnp.float32)])
        z, m_parts = _sc_zm(ad, asrc, srcp, dstp, gp)
        e, s_parts = _sc_es(m_parts, z, dstp)
        parts = _sc_agg(s_parts, e, srcp, dstp, h)
        return parts[0], parts[1]

    h1, a1 = _tc_project(x, W, b8, att2)
    p0, p1 = hop(x, h1, a1, gp_hops[0])
    f1, h2, a2 = _tc_project_res(x, p0[:N], p1[:N], W, b8, att2)
    q0, q1 = hop(f1, h2, a2, gp_hops[1])
    return _sc_root(q0, q1, f1, root_index)


# speedup vs baseline: 8.6063x; 1.1542x over previous
import functools
import jax
import jax.numpy as jnp
from jax import lax
from jax.experimental import pallas as pl
from jax.experimental.pallas import tpu as pltpu
from jax.experimental.pallas import tpu_sc as plsc

N = 10000
E = 320000
D = 128
TEMP = 0.1
ROOTS = 1024
N_HOPS = 2

NC = 2
NS = 16
NT = NC * NS
EPT = E // NT
CH = 64
NCH = 160
EPTP = NCH * CH
NP = 10240
NPS = NP // NS
G8 = CH // 16
NCHS = NCH // 5

_NEG = -1e30


def _seg_prefix(op, vals, keys, lane):
    out = vals
    for sh in (1, 2, 4, 8):
        idx = jnp.maximum(lane - sh, 0)
        vs = out.at[idx].get(mode="promise_in_bounds")
        ks = keys.at[idx].get(mode="promise_in_bounds")
        take = (ks == keys) & (lane >= sh)
        out = jnp.where(take, op(out, vs), out)
    return out


def _run_end(keys, lane):
    nidx = jnp.minimum(lane + 1, 15)
    knx = keys.at[nidx].get(mode="promise_in_bounds")
    return (knx != keys) | (lane == 15)


def _wid():
    return lax.axis_index("s") * NC + lax.axis_index("c")


def _sc_mesh():
    return plsc.VectorSubcoreMesh(core_axis_name="c", subcore_axis_name="s")


def _mm_body(f_ref, w_ref, b_ref, at_ref, h_ref, a_ref):
    h = jnp.dot(f_ref[...], w_ref[...], preferred_element_type=jnp.float32)
    h = h + b_ref[0:1, :]
    h_ref[...] = h
    a_ref[...] = jnp.dot(h, at_ref[...], preferred_element_type=jnp.float32)


def _mm_res_body(x_ref, p0_ref, p1_ref, w_ref, b_ref, at_ref,
                 f_ref, h_ref, a_ref):
    f = x_ref[...] + p0_ref[...] + p1_ref[...]
    f_ref[...] = f
    h = jnp.dot(f, w_ref[...], preferred_element_type=jnp.float32)
    h = h + b_ref[0:1, :]
    h_ref[...] = h
    a_ref[...] = jnp.dot(h, at_ref[...], preferred_element_type=jnp.float32)


_ROWS = 1000
_GRID = N // _ROWS


def _tc_project(f, W, b8, att2):
    return pl.pallas_call(
        _mm_body,
        grid=(_GRID,),
        in_specs=[
            pl.BlockSpec((_ROWS, D), lambda i: (i, 0)),
            pl.BlockSpec((D, D), lambda i: (0, 0)),
            pl.BlockSpec((8, D), lambda i: (0, 0)),
            pl.BlockSpec((D, D), lambda i: (0, 0)),
        ],
        out_specs=[
            pl.BlockSpec((_ROWS, D), lambda i: (i, 0)),
            pl.BlockSpec((_ROWS, D), lambda i: (i, 0)),
        ],
        out_shape=[
            jax.ShapeDtypeStruct((N, D), jnp.float32),
            jax.ShapeDtypeStruct((N, D), jnp.float32),
        ],
    )(f, W, b8, att2)


def _tc_project_res(x, p0, p1, W, b8, att2):
    return pl.pallas_call(
        _mm_res_body,
        grid=(_GRID,),
        in_specs=[
            pl.BlockSpec((_ROWS, D), lambda i: (i, 0)),
            pl.BlockSpec((_ROWS, D), lambda i: (i, 0)),
            pl.BlockSpec((_ROWS, D), lambda i: (i, 0)),
            pl.BlockSpec((D, D), lambda i: (0, 0)),
            pl.BlockSpec((8, D), lambda i: (0, 0)),
            pl.BlockSpec((D, D), lambda i: (0, 0)),
        ],
        out_specs=[
            pl.BlockSpec((_ROWS, D), lambda i: (i, 0)),
            pl.BlockSpec((_ROWS, D), lambda i: (i, 0)),
            pl.BlockSpec((_ROWS, D), lambda i: (i, 0)),
        ],
        out_shape=[
            jax.ShapeDtypeStruct((N, D), jnp.float32),
            jax.ShapeDtypeStruct((N, D), jnp.float32),
            jax.ShapeDtypeStruct((N, D), jnp.float32),
        ],
    )(x, p0, p1, W, b8, att2)


def _zm_body(ad_hbm, as_hbm, srcp, dstp, gp, z_out, m_parts,
             ad_t, as_t, src_t, dst_t, g_t, z_t, m_loc, mout, cmb, shm):
    c = lax.axis_index("c")
    s = lax.axis_index("s")
    wid = _wid()
    pltpu.sync_copy(ad_hbm, ad_t)
    pltpu.sync_copy(as_hbm, as_t)
    pltpu.sync_copy(srcp.at[wid], src_t)
    pltpu.sync_copy(dstp.at[wid], dst_t)
    pltpu.sync_copy(gp.at[wid], g_t)

    def init(i, _):
        m_loc[pl.ds(i * 16, 16)] = jnp.full((16,), _NEG, jnp.float32)
        return 0

    lax.fori_loop(0, NP // 16, init, 0)

    lane = lax.broadcasted_iota(jnp.int32, (16,), 0)

    def grp(r, _):
        for k in range(G8):
            sl = pl.ds(k * 16, 16)
            sv = src_t[r, sl]
            dv = dst_t[r, sl]
            gv = g_t[r, sl]
            av = plsc.load_gather(ad_t, [dv])
            bv = plsc.load_gather(as_t, [sv])
            t = av + bv
            lr = jnp.where(t >= 0.0, t, t * 0.2)
            z = (lr + gv) * (1.0 / TEMP)
            z_t[pl.ds(r * CH + k * 16, 16)] = z
            dk, zk = plsc.sort_key_val(dv, z)
            zc = _seg_prefix(jnp.maximum, zk, dk, lane)
            end = _run_end(dk, lane)
            mo = plsc.load_gather(m_loc, [dk])
            plsc.store_scatter(m_loc, [dk], jnp.maximum(mo, zc), mask=end)
        return 0

    lax.fori_loop(0, NCH, grp, 0)
    pltpu.sync_copy(z_t, z_out.at[wid])

    pltpu.sync_copy(m_loc, shm.at[s])
    plsc.subcore_barrier()
    pltpu.sync_copy(shm.at[:, pl.ds(s * NPS, NPS)], cmb)

    def red(i, _):
        sl = pl.ds(i * 16, 16)
        acc = cmb[0, sl]
        for t in range(1, NS):
            acc = jnp.maximum(acc, cmb[t, sl])
        mout[sl] = acc
        return 0

    lax.fori_loop(0, NPS // 16, red, 0)
    pltpu.sync_copy(mout, m_parts.at[c, pl.ds(s * NPS, NPS)])


def _sc_zm(ad, asrc, srcp, dstp, gp):
    return pl.kernel(
        _zm_body,
        out_type=[
            jax.ShapeDtypeStruct((NT, EPTP), jnp.float32),
            jax.ShapeDtypeStruct((NC, NP), jnp.float32),
        ],
        mesh=_sc_mesh(),
        compiler_params=pltpu.CompilerParams(needs_layout_passes=False),
        scratch_types=[
            pltpu.VMEM((NP,), jnp.float32),
            pltpu.VMEM((NP,), jnp.float32),
            pltpu.VMEM((NCH, CH), jnp.int32),
            pltpu.VMEM((NCH, CH), jnp.int32),
            pltpu.VMEM((NCH, CH), jnp.float32),
            pltpu.VMEM((EPTP,), jnp.float32),
            pltpu.VMEM((NP,), jnp.float32),
            pltpu.VMEM((NPS,), jnp.float32),
            pltpu.VMEM((NS, NPS), jnp.float32),
            pltpu.VMEM_SHARED((NS, NP), jnp.float32),
        ],
    )(ad, asrc, srcp, dstp, gp)


def _es_body(m_parts, z_in, dstp, s_parts,
             m_a, m_b, z_t, dst_t, s_loc, sout, cmb, shm):
    c = lax.axis_index("c")
    s = lax.axis_index("s")
    wid = _wid()
    pltpu.sync_copy(m_parts.at[0], m_a)
    pltpu.sync_copy(m_parts.at[1], m_b)
    pltpu.sync_copy(z_in.at[wid], z_t)
    pltpu.sync_copy(dstp.at[wid], dst_t)

    def init(i, _):
        sl = pl.ds(i * 16, 16)
        m_a[sl] = jnp.maximum(m_a[sl], m_b[sl])
        s_loc[sl] = jnp.zeros((16,), jnp.float32)
        return 0

    lax.fori_loop(0, NP // 16, init, 0)

    lane = lax.broadcasted_iota(jnp.int32, (16,), 0)

    def grp(r, _):
        for k in range(G8):
            sl = pl.ds(k * 16, 16)
            zv = z_t[pl.ds(r * CH + k * 16, 16)]
            dv = dst_t[r, sl]
            mv = plsc.load_gather(m_a, [dv])
            ev = jnp.exp(zv - mv)
            dk, ek = plsc.sort_key_val(dv, ev)
            ec = _seg_prefix(jnp.add, ek, dk, lane)
            end = _run_end(dk, lane)
            so = plsc.load_gather(s_loc, [dk])
            plsc.store_scatter(s_loc, [dk], so + ec, mask=end)
        return 0

    lax.fori_loop(0, NCH, grp, 0)

    pltpu.sync_copy(s_loc, shm.at[s])
    plsc.subcore_barrier()
    pltpu.sync_copy(shm.at[:, pl.ds(s * NPS, NPS)], cmb)

    def red(i, _):
        sl = pl.ds(i * 16, 16)
        acc = cmb[0, sl]
        for t in range(1, NS):
            acc = acc + cmb[t, sl]
        sout[sl] = acc
        return 0

    lax.fori_loop(0, NPS // 16, red, 0)
    pltpu.sync_copy(sout, s_parts.at[c, pl.ds(s * NPS, NPS)])


def _sc_es(m_parts, z_in, dstp):
    return pl.kernel(
        _es_body,
        out_type=jax.ShapeDtypeStruct((NC, NP), jnp.float32),
        mesh=_sc_mesh(),
        compiler_params=pltpu.CompilerParams(needs_layout_passes=False),
        scratch_types=[
            pltpu.VMEM((NP,), jnp.float32),
            pltpu.VMEM((NP,), jnp.float32),
            pltpu.VMEM((EPTP,), jnp.float32),
            pltpu.VMEM((NCH, CH), jnp.int32),
            pltpu.VMEM((NP,), jnp.float32),
            pltpu.VMEM((NPS,), jnp.float32),
            pltpu.VMEM((NS, NPS), jnp.float32),
            pltpu.VMEM_SHARED((NS, NP), jnp.float32),
        ],
    )(m_parts, z_in, dstp)


def _al_body(m_parts, s_parts, z_in, dstp, al_out,
             m_a, m_b, s_tmp, z_t, dst_t, al_t):
    wid = _wid()
    pltpu.sync_copy(m_parts.at[0], m_a)
    pltpu.sync_copy(m_parts.at[1], m_b)
    pltpu.sync_copy(z_in.at[wid], z_t)
    pltpu.sync_copy(dstp.at[wid], dst_t)

    def initm(i, _):
        sl = pl.ds(i * 16, 16)
        m_a[sl] = jnp.maximum(m_a[sl], m_b[sl])
        return 0

    lax.fori_loop(0, NP // 16, initm, 0)
    pltpu.sync_copy(s_parts.at[0], m_b)
    for q in range(NP // NPS):
        pltpu.sync_copy(s_parts.at[1, pl.ds(q * NPS, NPS)], s_tmp)

        def inits(i, _):
            sl = pl.ds(i * 16, 16)
            m_b[pl.ds(q * NPS + i * 16, 16)] = (
                m_b[pl.ds(q * NPS + i * 16, 16)] + s_tmp[sl])
            return 0

        lax.fori_loop(0, NPS // 16, inits, 0)

    lane = lax.broadcasted_iota(jnp.int32, (16,), 0)

    def grp(r, _):
        for k in range(G8):
            fl = pl.ds(r * CH + k * 16, 16)
            zv = z_t[fl]
            dv = dst_t[r, pl.ds(k * 16, 16)]
            mv = plsc.load_gather(m_a, [dv])
            sv = plsc.load_gather(m_b, [dv])
            av = jnp.exp(zv - mv) / (sv + 1e-16)
            pos = r * CH + k * 16 + lane
            av = jnp.where(pos < EPT, av, 0.0)
            al_t[fl] = av
        return 0

    lax.fori_loop(0, NCH, grp, 0)
    pltpu.sync_copy(al_t, al_out.at[wid])


def _sc_al(m_parts, s_parts, z_in, dstp):
    return pl.kernel(
        _al_body,
        out_type=jax.ShapeDtypeStruct((NT, EPTP), jnp.float32),
        mesh=_sc_mesh(),
        compiler_params=pltpu.CompilerParams(needs_layout_passes=False),
        scratch_types=[
            pltpu.VMEM((NP,), jnp.float32),
            pltpu.VMEM((NP,), jnp.float32),
            pltpu.VMEM((NPS,), jnp.float32),
            pltpu.VMEM((EPTP,), jnp.float32),
            pltpu.VMEM((NCH, CH), jnp.int32),
            pltpu.VMEM((EPTP,), jnp.float32),
        ],
    )(m_parts, s_parts, z_in, dstp)


def _agg_body(al_in, srcp, dstp, h_hbm, out_parts,
              src_t, al_h, dst2_h, rows0, rows1, out_sh, sem0, sem1):
    c = lax.axis_index("c")
    s = lax.axis_index("s")
    wid = _wid()
    pltpu.sync_copy(srcp.at[wid], src_t)

    def zero_rows(r, _):
        for dcol in range(D // 16):
            rows0[r, pl.ds(dcol * 16, 16)] = jnp.zeros((16,), jnp.float32)
        return 0

    lax.fori_loop(0, CH, zero_rows, 0)
    for q in range(NPS // CH):
        pltpu.sync_copy(rows0, out_sh.at[pl.ds(s * NPS + q * CH, CH)])
    plsc.subcore_barrier()

    bufs = (rows0, rows1)
    sems = (sem0, sem1)

    def do_chunk(rl, rows):
        def grp(g, _):
            av = al_h[pl.ds(rl * CH + g * 16, 16)]
            for j in range(16):
                asp = av.at[jnp.full((16,), j, jnp.int32)].get(
                    mode="promise_in_bounds")
                for dcol in range(D // 16):
                    dsl = pl.ds(dcol * 16, 16)
                    rows[g * 16 + j, dsl] = rows[g * 16 + j, dsl] * asp
            return 0

        lax.fori_loop(0, G8, grp, 0)
        pltpu.sync_copy(rows, out_sh.at[dst2_h.at[rl, 0]], add=True)

    for piece in range(NCH // NCHS):
        pltpu.sync_copy(al_in.at[wid, pl.ds(piece * NCHS * CH, NCHS * CH)],
                        al_h)
        pltpu.sync_copy(dstp.at[wid, pl.ds(piece * NCHS, NCHS)],
                        dst2_h.at[:, 0])
        pltpu.async_copy(
            h_hbm.at[src_t.at[piece * NCHS]], bufs[0], sems[0])

        def pair(rr, _):
            for p in (0, 1):
                rl = rr * 2 + p
                nxt = jnp.minimum(rl + 1, NCHS - 1)
                pltpu.async_copy(
                    h_hbm.at[src_t.at[piece * NCHS + nxt]],
                    bufs[1 - p], sems[1 - p])
                pltpu.make_async_copy(
                    h_hbm.at[src_t.at[piece * NCHS + rl]],
                    bufs[p], sems[p]).wait()
                do_chunk(rl, bufs[p])
            return 0

        lax.fori_loop(0, NCHS // 2, pair, 0)
        pltpu.make_async_copy(
            h_hbm.at[src_t.at[piece * NCHS + NCHS - 1]],
            bufs[0], sems[0]).wait()

    plsc.subcore_barrier()
    pltpu.sync_copy(out_sh.at[pl.ds(s * NPS, NPS)],
                    out_parts.at[c, pl.ds(s * NPS, NPS)])


def _sc_agg(al_in, srcp, dstp, h):
    return pl.kernel(
        _agg_body,
        out_type=jax.ShapeDtypeStruct((NC, NP, D), jnp.float32),
        mesh=_sc_mesh(),
        compiler_params=pltpu.CompilerParams(needs_layout_passes=False),
        scratch_types=[
            pltpu.VMEM((NCH, CH), jnp.int32),
            pltpu.VMEM((NCHS * CH,), jnp.float32),
            pltpu.VMEM((NCHS, 2, CH), jnp.int32),
            pltpu.VMEM((CH, D), jnp.float32),
            pltpu.VMEM((CH, D), jnp.float32),
            pltpu.VMEM_SHARED((NP, D), jnp.float32),
            pltpu.SemaphoreType.DMA,
            pltpu.SemaphoreType.DMA,
        ],
    )(al_in, srcp, dstp, h)


_RPT = ROOTS // NT


def _root_body(p0, p1, f1, roots, out, ridx, r0, r1, r2, o, sem):
    wid = _wid()
    sl = pl.ds(wid * _RPT, _RPT)
    pltpu.sync_copy(roots.at[sl], ridx)
    pltpu.async_copy(p0.at[ridx], r0, sem).wait()
    pltpu.async_copy(p1.at[ridx], r1, sem).wait()
    pltpu.async_copy(f1.at[ridx], r2, sem).wait()

    def add(i, _):
        for dcol in range(D // 16):
            dsl = pl.ds(dcol * 16, 16)
            o[i, dsl] = r0[i, dsl] + r1[i, dsl] + r2[i, dsl]
        return 0

    lax.fori_loop(0, _RPT, add, 0)
    pltpu.sync_copy(o, out.at[sl])


def _sc_root(p0, p1, f1, roots):
    return pl.kernel(
        _root_body,
        out_type=jax.ShapeDtypeStruct((ROOTS, D), jnp.float32),
        mesh=_sc_mesh(),
        compiler_params=pltpu.CompilerParams(needs_layout_passes=False),
        scratch_types=[
            pltpu.VMEM((_RPT,), jnp.int32),
            pltpu.VMEM((_RPT, D), jnp.float32),
            pltpu.VMEM((_RPT, D), jnp.float32),
            pltpu.VMEM((_RPT, D), jnp.float32),
            pltpu.VMEM((_RPT, D), jnp.float32),
            pltpu.SemaphoreType.DMA,
        ],
    )(p0, p1, f1, roots)


def _to_tiles(a, pad_val):
    a2 = a.reshape(NT, EPT)
    pad = jnp.full((NT, EPTP - EPT), pad_val, a2.dtype)
    return jnp.concatenate([a2, pad], axis=1).reshape(NT, NCH, CH)


def kernel(x, edge_index, root_index, W, b, att):
    key = jax.random.key(42)
    gp_hops = []
    for i in range(N_HOPS):
        u = jax.random.uniform(jax.random.fold_in(key, i), (E,),
                               minval=1e-6, maxval=1.0 - 1e-6)
        gp_hops.append(_to_tiles(-jnp.log(-jnp.log(u)), _NEG))

    srcp = _to_tiles(edge_index[0], 0)
    dstp = _to_tiles(edge_index[1], 0)
    b8 = jnp.broadcast_to(b.reshape(1, D), (8, D))
    att2 = jnp.zeros((D, D), jnp.float32)
    att2 = att2.at[:, 0].set(att[:D]).at[:, 1].set(att[D:])

    def hop(f, h, a, gp):
        ad = jnp.concatenate([a[:, 0], jnp.zeros((NP - N,), jnp.float32)])
        asrc = jnp.concatenate([a[:, 1], jnp.zeros((NP - N,), jnp.float32)])
        z, m_parts = _sc_zm(ad, asrc, srcp, dstp, gp)
        s_parts = _sc_es(m_parts, z, dstp)
        al = _sc_al(m_parts, s_parts, z, dstp)
        parts = _sc_agg(al, srcp, dstp, h)
        return parts[0], parts[1]

    h1, a1 = _tc_project(x, W, b8, att2)
    p0, p1 = hop(x, h1, a1, gp_hops[0])
    f1, h2, a2 = _tc_project_res(x, p0[:N], p1[:N], W, b8, att2)
    q0, q1 = hop(f1, h2, a2, gp_hops[1])
    return _sc_root(q0, q1, f1, root_index)


# speedup vs baseline: 8.8142x; 1.0242x over previous
import functools
import jax
import jax.numpy as jnp
from jax import lax
from jax.experimental import pallas as pl
from jax.experimental.pallas import tpu as pltpu
from jax.experimental.pallas import tpu_sc as plsc

N = 10000
E = 320000
D = 128
TEMP = 0.1
ROOTS = 1024
N_HOPS = 2

NC = 2
NS = 16
NT = NC * NS
EPT = E // NT
CH = 64
NCH = 160
EPTP = NCH * CH
NP = 10240
NPS = NP // NS
G8 = CH // 16
NCHS = NCH // 5

_NEG = -1e30


def _seg_prefix(op, vals, keys, lane):
    out = vals
    for sh in (1, 2, 4, 8):
        idx = jnp.maximum(lane - sh, 0)
        vs = out.at[idx].get(mode="promise_in_bounds")
        ks = keys.at[idx].get(mode="promise_in_bounds")
        take = (ks == keys) & (lane >= sh)
        out = jnp.where(take, op(out, vs), out)
    return out


def _run_end(keys, lane):
    nidx = jnp.minimum(lane + 1, 15)
    knx = keys.at[nidx].get(mode="promise_in_bounds")
    return (knx != keys) | (lane == 15)


def _wid():
    return lax.axis_index("s") * NC + lax.axis_index("c")


def _sc_mesh():
    return plsc.VectorSubcoreMesh(core_axis_name="c", subcore_axis_name="s")


def _mm_body(f_ref, w_ref, b_ref, at_ref, pz_ref, h_ref, a_ref, h16_ref):
    h = jnp.dot(f_ref[...], w_ref[...], preferred_element_type=jnp.float32)
    h = h + b_ref[0:1, :]
    h_ref[...] = h
    a_ref[...] = jnp.dot(h, at_ref[...], preferred_element_type=jnp.float32)
    hsw = jnp.dot(h, pz_ref[...], preferred_element_type=jnp.float32)
    h16_ref[...] = hsw.astype(jnp.bfloat16)


def _mm_res_body(x_ref, p0_ref, p1_ref, w_ref, b_ref, at_ref, pz_ref,
                 f_ref, h_ref, a_ref, h16_ref):
    f = x_ref[...] + p0_ref[...] + p1_ref[...]
    f_ref[...] = f
    h = jnp.dot(f, w_ref[...], preferred_element_type=jnp.float32)
    h = h + b_ref[0:1, :]
    h_ref[...] = h
    a_ref[...] = jnp.dot(h, at_ref[...], preferred_element_type=jnp.float32)
    hsw = jnp.dot(h, pz_ref[...], preferred_element_type=jnp.float32)
    h16_ref[...] = hsw.astype(jnp.bfloat16)


_ROWS = 1000
_GRID = N // _ROWS


def _tc_project(f, W, b8, att2, pz):
    return pl.pallas_call(
        _mm_body,
        grid=(_GRID,),
        in_specs=[
            pl.BlockSpec((_ROWS, D), lambda i: (i, 0)),
            pl.BlockSpec((D, D), lambda i: (0, 0)),
            pl.BlockSpec((8, D), lambda i: (0, 0)),
            pl.BlockSpec((D, D), lambda i: (0, 0)),
            pl.BlockSpec((D, D), lambda i: (0, 0)),
        ],
        out_specs=[
            pl.BlockSpec((_ROWS, D), lambda i: (i, 0)),
            pl.BlockSpec((_ROWS, D), lambda i: (i, 0)),
            pl.BlockSpec((_ROWS, D), lambda i: (i, 0)),
        ],
        out_shape=[
            jax.ShapeDtypeStruct((N, D), jnp.float32),
            jax.ShapeDtypeStruct((N, D), jnp.float32),
            jax.ShapeDtypeStruct((N, D), jnp.bfloat16),
        ],
    )(f, W, b8, att2, pz)


def _tc_project_res(x, p0, p1, W, b8, att2, pz):
    return pl.pallas_call(
        _mm_res_body,
        grid=(_GRID,),
        in_specs=[
            pl.BlockSpec((_ROWS, D), lambda i: (i, 0)),
            pl.BlockSpec((_ROWS, D), lambda i: (i, 0)),
            pl.BlockSpec((_ROWS, D), lambda i: (i, 0)),
            pl.BlockSpec((D, D), lambda i: (0, 0)),
            pl.BlockSpec((8, D), lambda i: (0, 0)),
            pl.BlockSpec((D, D), lambda i: (0, 0)),
            pl.BlockSpec((D, D), lambda i: (0, 0)),
        ],
        out_specs=[
            pl.BlockSpec((_ROWS, D), lambda i: (i, 0)),
            pl.BlockSpec((_ROWS, D), lambda i: (i, 0)),
            pl.BlockSpec((_ROWS, D), lambda i: (i, 0)),
            pl.BlockSpec((_ROWS, D), lambda i: (i, 0)),
        ],
        out_shape=[
            jax.ShapeDtypeStruct((N, D), jnp.float32),
            jax.ShapeDtypeStruct((N, D), jnp.float32),
            jax.ShapeDtypeStruct((N, D), jnp.float32),
            jax.ShapeDtypeStruct((N, D), jnp.bfloat16),
        ],
    )(x, p0, p1, W, b8, att2, pz)


def _zm_body(ad_hbm, as_hbm, srcp, dstp, gp, z_out, m_parts,
             ad_t, as_t, src_t, dst_t, g_t, z_t, m_loc, mout, cmb, shm):
    c = lax.axis_index("c")
    s = lax.axis_index("s")
    wid = _wid()
    pltpu.sync_copy(ad_hbm, ad_t)
    pltpu.sync_copy(as_hbm, as_t)
    pltpu.sync_copy(srcp.at[wid], src_t)
    pltpu.sync_copy(dstp.at[wid], dst_t)
    pltpu.sync_copy(gp.at[wid], g_t)

    def init(i, _):
        m_loc[pl.ds(i * 16, 16)] = jnp.full((16,), _NEG, jnp.float32)
        return 0

    lax.fori_loop(0, NP // 16, init, 0)

    lane = lax.broadcasted_iota(jnp.int32, (16,), 0)

    def grp(r, _):
        for k in range(G8):
            sl = pl.ds(k * 16, 16)
            sv = src_t[r, sl]
            dv = dst_t[r, sl]
            gv = g_t[r, sl]
            av = plsc.load_gather(ad_t, [dv])
            bv = plsc.load_gather(as_t, [sv])
            t = av + bv
            lr = jnp.where(t >= 0.0, t, t * 0.2)
            z = (lr + gv) * (1.0 / TEMP)
            z_t[pl.ds(r * CH + k * 16, 16)] = z
            dk, zk = plsc.sort_key_val(dv, z)
            zc = _seg_prefix(jnp.maximum, zk, dk, lane)
            end = _run_end(dk, lane)
            mo = plsc.load_gather(m_loc, [dk])
            plsc.store_scatter(m_loc, [dk], jnp.maximum(mo, zc), mask=end)
        return 0

    lax.fori_loop(0, NCH, grp, 0)
    pltpu.sync_copy(z_t, z_out.at[wid])

    pltpu.sync_copy(m_loc, shm.at[s])
    plsc.subcore_barrier()
    pltpu.sync_copy(shm.at[:, pl.ds(s * NPS, NPS)], cmb)

    def red(i, _):
        sl = pl.ds(i * 16, 16)
        acc = cmb[0, sl]
        for t in range(1, NS):
            acc = jnp.maximum(acc, cmb[t, sl])
        mout[sl] = acc
        return 0

    lax.fori_loop(0, NPS // 16, red, 0)
    pltpu.sync_copy(mout, m_parts.at[c, pl.ds(s * NPS, NPS)])


def _sc_zm(ad, asrc, srcp, dstp, gp):
    return pl.kernel(
        _zm_body,
        out_type=[
            jax.ShapeDtypeStruct((NT, EPTP), jnp.float32),
            jax.ShapeDtypeStruct((NC, NP), jnp.float32),
        ],
        mesh=_sc_mesh(),
        compiler_params=pltpu.CompilerParams(needs_layout_passes=False),
        scratch_types=[
            pltpu.VMEM((NP,), jnp.float32),
            pltpu.VMEM((NP,), jnp.float32),
            pltpu.VMEM((NCH, CH), jnp.int32),
            pltpu.VMEM((NCH, CH), jnp.int32),
            pltpu.VMEM((NCH, CH), jnp.float32),
            pltpu.VMEM((EPTP,), jnp.float32),
            pltpu.VMEM((NP,), jnp.float32),
            pltpu.VMEM((NPS,), jnp.float32),
            pltpu.VMEM((NS, NPS), jnp.float32),
            pltpu.VMEM_SHARED((NS, NP), jnp.float32),
        ],
    )(ad, asrc, srcp, dstp, gp)


def _es_body(m_parts, z_in, dstp, s_parts,
             m_a, m_b, z_t, dst_t, s_loc, sout, cmb, shm):
    c = lax.axis_index("c")
    s = lax.axis_index("s")
    wid = _wid()
    pltpu.sync_copy(m_parts.at[0], m_a)
    pltpu.sync_copy(m_parts.at[1], m_b)
    pltpu.sync_copy(z_in.at[wid], z_t)
    pltpu.sync_copy(dstp.at[wid], dst_t)

    def init(i, _):
        sl = pl.ds(i * 16, 16)
        m_a[sl] = jnp.maximum(m_a[sl], m_b[sl])
        s_loc[sl] = jnp.zeros((16,), jnp.float32)
        return 0

    lax.fori_loop(0, NP // 16, init, 0)

    lane = lax.broadcasted_iota(jnp.int32, (16,), 0)

    def grp(r, _):
        for k in range(G8):
            sl = pl.ds(k * 16, 16)
            zv = z_t[pl.ds(r * CH + k * 16, 16)]
            dv = dst_t[r, sl]
            mv = plsc.load_gather(m_a, [dv])
            ev = jnp.exp(zv - mv)
            dk, ek = plsc.sort_key_val(dv, ev)
            ec = _seg_prefix(jnp.add, ek, dk, lane)
            end = _run_end(dk, lane)
            so = plsc.load_gather(s_loc, [dk])
            plsc.store_scatter(s_loc, [dk], so + ec, mask=end)
        return 0

    lax.fori_loop(0, NCH, grp, 0)

    pltpu.sync_copy(s_loc, shm.at[s])
    plsc.subcore_barrier()
    pltpu.sync_copy(shm.at[:, pl.ds(s * NPS, NPS)], cmb)

    def red(i, _):
        sl = pl.ds(i * 16, 16)
        acc = cmb[0, sl]
        for t in range(1, NS):
            acc = acc + cmb[t, sl]
        sout[sl] = acc
        return 0

    lax.fori_loop(0, NPS // 16, red, 0)
    pltpu.sync_copy(sout, s_parts.at[c, pl.ds(s * NPS, NPS)])


def _sc_es(m_parts, z_in, dstp):
    return pl.kernel(
        _es_body,
        out_type=jax.ShapeDtypeStruct((NC, NP), jnp.float32),
        mesh=_sc_mesh(),
        compiler_params=pltpu.CompilerParams(needs_layout_passes=False),
        scratch_types=[
            pltpu.VMEM((NP,), jnp.float32),
            pltpu.VMEM((NP,), jnp.float32),
            pltpu.VMEM((EPTP,), jnp.float32),
            pltpu.VMEM((NCH, CH), jnp.int32),
            pltpu.VMEM((NP,), jnp.float32),
            pltpu.VMEM((NPS,), jnp.float32),
            pltpu.VMEM((NS, NPS), jnp.float32),
            pltpu.VMEM_SHARED((NS, NP), jnp.float32),
        ],
    )(m_parts, z_in, dstp)


def _al_body(m_parts, s_parts, z_in, dstp, al_out,
             m_a, m_b, s_tmp, z_t, dst_t, al_t):
    wid = _wid()
    pltpu.sync_copy(m_parts.at[0], m_a)
    pltpu.sync_copy(m_parts.at[1], m_b)
    pltpu.sync_copy(z_in.at[wid], z_t)
    pltpu.sync_copy(dstp.at[wid], dst_t)

    def initm(i, _):
        sl = pl.ds(i * 16, 16)
        m_a[sl] = jnp.maximum(m_a[sl], m_b[sl])
        return 0

    lax.fori_loop(0, NP // 16, initm, 0)
    pltpu.sync_copy(s_parts.at[0], m_b)
    for q in range(NP // NPS):
        pltpu.sync_copy(s_parts.at[1, pl.ds(q * NPS, NPS)], s_tmp)

        def inits(i, _):
            sl = pl.ds(i * 16, 16)
            m_b[pl.ds(q * NPS + i * 16, 16)] = (
                m_b[pl.ds(q * NPS + i * 16, 16)] + s_tmp[sl])
            return 0

        lax.fori_loop(0, NPS // 16, inits, 0)

    lane = lax.broadcasted_iota(jnp.int32, (16,), 0)

    def grp(r, _):
        for k in range(G8):
            fl = pl.ds(r * CH + k * 16, 16)
            zv = z_t[fl]
            dv = dst_t[r, pl.ds(k * 16, 16)]
            mv = plsc.load_gather(m_a, [dv])
            sv = plsc.load_gather(m_b, [dv])
            av = jnp.exp(zv - mv) / (sv + 1e-16)
            pos = r * CH + k * 16 + lane
            av = jnp.where(pos < EPT, av, 0.0)
            al_t[fl] = av
        return 0

    lax.fori_loop(0, NCH, grp, 0)
    pltpu.sync_copy(al_t, al_out.at[wid])


def _sc_al(m_parts, s_parts, z_in, dstp):
    return pl.kernel(
        _al_body,
        out_type=jax.ShapeDtypeStruct((NT, EPTP), jnp.float32),
        mesh=_sc_mesh(),
        compiler_params=pltpu.CompilerParams(needs_layout_passes=False),
        scratch_types=[
            pltpu.VMEM((NP,), jnp.float32),
            pltpu.VMEM((NP,), jnp.float32),
            pltpu.VMEM((NPS,), jnp.float32),
            pltpu.VMEM((EPTP,), jnp.float32),
            pltpu.VMEM((NCH, CH), jnp.int32),
            pltpu.VMEM((EPTP,), jnp.float32),
        ],
    )(m_parts, s_parts, z_in, dstp)


def _agg_body(al_in, srcp, dstp, h16_hbm, out_parts,
              src_t, al_h, dst2_h, rows0, rows1, rowsf, out_sh, sem0, sem1):
    c = lax.axis_index("c")
    s = lax.axis_index("s")
    wid = _wid()
    pltpu.sync_copy(srcp.at[wid], src_t)

    def zero_rows(r, _):
        for dcol in range(D // 16):
            rowsf[r, pl.ds(dcol * 16, 16)] = jnp.zeros((16,), jnp.float32)
        return 0

    lax.fori_loop(0, CH, zero_rows, 0)
    for q in range(NPS // CH):
        pltpu.sync_copy(rowsf, out_sh.at[pl.ds(s * NPS + q * CH, CH)])
    plsc.subcore_barrier()

    bufs = (rows0, rows1)
    sems = (sem0, sem1)
    himask = jnp.full((16,), -65536, jnp.int32)

    def do_chunk(rl, rows):
        def grp(g, _):
            av = al_h[pl.ds(rl * CH + g * 16, 16)]
            for j in range(16):
                asp = av.at[jnp.full((16,), j, jnp.int32)].get(
                    mode="promise_in_bounds")
                for d2 in range(D // 32):
                    vi = rows[g * 16 + j, pl.ds(d2 * 16, 16)]
                    lo = plsc.bitcast(vi << 16, jnp.float32)
                    hi = plsc.bitcast(vi & himask, jnp.float32)
                    rowsf[g * 16 + j, pl.ds(d2 * 32, 16)] = lo * asp
                    rowsf[g * 16 + j, pl.ds(d2 * 32 + 16, 16)] = hi * asp
            return 0

        lax.fori_loop(0, G8, grp, 0)
        pltpu.sync_copy(rowsf, out_sh.at[dst2_h.at[rl, 0]], add=True)

    for piece in range(NCH // NCHS):
        pltpu.sync_copy(al_in.at[wid, pl.ds(piece * NCHS * CH, NCHS * CH)],
                        al_h)
        pltpu.sync_copy(dstp.at[wid, pl.ds(piece * NCHS, NCHS)],
                        dst2_h.at[:, 0])
        pltpu.async_copy(
            h16_hbm.at[src_t.at[piece * NCHS]], bufs[0], sems[0])

        def pair(rr, _):
            for p in (0, 1):
                rl = rr * 2 + p
                nxt = jnp.minimum(rl + 1, NCHS - 1)
                pltpu.async_copy(
                    h16_hbm.at[src_t.at[piece * NCHS + nxt]],
                    bufs[1 - p], sems[1 - p])
                pltpu.make_async_copy(
                    h16_hbm.at[src_t.at[piece * NCHS + rl]],
                    bufs[p], sems[p]).wait()
                do_chunk(rl, bufs[p])
            return 0

        lax.fori_loop(0, NCHS // 2, pair, 0)
        pltpu.make_async_copy(
            h16_hbm.at[src_t.at[piece * NCHS + NCHS - 1]],
            bufs[0], sems[0]).wait()

    plsc.subcore_barrier()
    pltpu.sync_copy(out_sh.at[pl.ds(s * NPS, NPS)],
                    out_parts.at[c, pl.ds(s * NPS, NPS)])


def _sc_agg(al_in, srcp, dstp, h16):
    return pl.kernel(
        _agg_body,
        out_type=jax.ShapeDtypeStruct((NC, NP, D), jnp.float32),
        mesh=_sc_mesh(),
        compiler_params=pltpu.CompilerParams(
            needs_layout_passes=False, use_tc_tiling_on_sc=False),
        scratch_types=[
            pltpu.VMEM((NCH, CH), jnp.int32),
            pltpu.VMEM((NCHS * CH,), jnp.float32),
            pltpu.VMEM((NCHS, 2, CH), jnp.int32),
            pltpu.VMEM((CH, D // 2), jnp.int32),
            pltpu.VMEM((CH, D // 2), jnp.int32),
            pltpu.VMEM((CH, D), jnp.float32),
            pltpu.VMEM_SHARED((NP, D), jnp.float32),
            pltpu.SemaphoreType.DMA,
            pltpu.SemaphoreType.DMA,
        ],
    )(al_in, srcp, dstp, h16)


_RPT = ROOTS // NT


def _root_body(p0, p1, f1, roots, out, ridx, r0, r1, r2, o, sem):
    wid = _wid()
    sl = pl.ds(wid * _RPT, _RPT)
    pltpu.sync_copy(roots.at[sl], ridx)
    pltpu.async_copy(p0.at[ridx], r0, sem).wait()
    pltpu.async_copy(p1.at[ridx], r1, sem).wait()
    pltpu.async_copy(f1.at[ridx], r2, sem).wait()

    def add(i, _):
        for dcol in range(D // 16):
            dsl = pl.ds(dcol * 16, 16)
            o[i, dsl] = r0[i, dsl] + r1[i, dsl] + r2[i, dsl]
        return 0

    lax.fori_loop(0, _RPT, add, 0)
    pltpu.sync_copy(o, out.at[sl])


def _sc_root(p0, p1, f1, roots):
    return pl.kernel(
        _root_body,
        out_type=jax.ShapeDtypeStruct((ROOTS, D), jnp.float32),
        mesh=_sc_mesh(),
        compiler_params=pltpu.CompilerParams(needs_layout_passes=False),
        scratch_types=[
            pltpu.VMEM((_RPT,), jnp.int32),
            pltpu.VMEM((_RPT, D), jnp.float32),
            pltpu.VMEM((_RPT, D), jnp.float32),
            pltpu.VMEM((_RPT, D), jnp.float32),
            pltpu.VMEM((_RPT, D), jnp.float32),
            pltpu.SemaphoreType.DMA,
        ],
    )(p0, p1, f1, roots)


def _to_tiles(a, pad_val):
    a2 = a.reshape(NT, EPT)
    pad = jnp.full((NT, EPTP - EPT), pad_val, a2.dtype)
    return jnp.concatenate([a2, pad], axis=1).reshape(NT, NCH, CH)


def kernel(x, edge_index, root_index, W, b, att):
    key = jax.random.key(42)
    gp_hops = []
    for i in range(N_HOPS):
        u = jax.random.uniform(jax.random.fold_in(key, i), (E,),
                               minval=1e-6, maxval=1.0 - 1e-6)
        gp_hops.append(_to_tiles(-jnp.log(-jnp.log(u)), _NEG))

    srcp = _to_tiles(edge_index[0], 0)
    dstp = _to_tiles(edge_index[1], 0)
    b8 = jnp.broadcast_to(b.reshape(1, D), (8, D))
    att2 = jnp.zeros((D, D), jnp.float32)
    att2 = att2.at[:, 0].set(att[:D]).at[:, 1].set(att[D:])
    import numpy as _np
    perm_src = _np.zeros((D,), _np.int32)
    for d2 in range(D // 32):
        for j in range(16):
            perm_src[d2 * 32 + 2 * j] = d2 * 32 + j
            perm_src[d2 * 32 + 2 * j + 1] = d2 * 32 + 16 + j
    pz = jnp.zeros((D, D), jnp.float32)
    pz = pz.at[jnp.asarray(perm_src), jnp.arange(D)].set(1.0)

    def hop(h, a, gp, h16):
        ad = jnp.concatenate([a[:, 0], jnp.zeros((NP - N,), jnp.float32)])
        asrc = jnp.concatenate([a[:, 1], jnp.zeros((NP - N,), jnp.float32)])
        z, m_parts = _sc_zm(ad, asrc, srcp, dstp, gp)
        s_parts = _sc_es(m_parts, z, dstp)
        al = _sc_al(m_parts, s_parts, z, dstp)
        parts = _sc_agg(al, srcp, dstp, h16)
        return parts[0], parts[1]

    def as_i32(h16):
        return jax.lax.bitcast_convert_type(
            h16.reshape(N, D // 2, 2), jnp.int32)

    h1, a1, h16_1 = _tc_project(x, W, b8, att2, pz)
    p0, p1 = hop(h1, a1, gp_hops[0], as_i32(h16_1))
    f1, h2, a2, h16_2 = _tc_project_res(x, p0[:N], p1[:N], W, b8, att2, pz)
    q0, q1 = hop(h2, a2, gp_hops[1], as_i32(h16_2))
    return _sc_root(q0, q1, f1, root_index)


# speedup vs baseline: 9.5602x; 1.0846x over previous
import functools
import jax
import jax.numpy as jnp
from jax import lax
from jax.experimental import pallas as pl
from jax.experimental.pallas import tpu as pltpu
from jax.experimental.pallas import tpu_sc as plsc

N = 10000
E = 320000
D = 128
TEMP = 0.1
ROOTS = 1024
N_HOPS = 2

NC = 2
NS = 16
NT = NC * NS
EPT = E // NT
CH = 64
NCH = 160
EPTP = NCH * CH
NP = 10240
NPS = NP // NS
G8 = CH // 16
NCHS = NCH // 5

_NEG = -1e30


def _seg_prefix(op, vals, keys, lane):
    out = vals
    for sh in (1, 2, 4, 8):
        idx = jnp.maximum(lane - sh, 0)
        vs = out.at[idx].get(mode="promise_in_bounds")
        ks = keys.at[idx].get(mode="promise_in_bounds")
        take = (ks == keys) & (lane >= sh)
        out = jnp.where(take, op(out, vs), out)
    return out


def _run_end(keys, lane):
    nidx = jnp.minimum(lane + 1, 15)
    knx = keys.at[nidx].get(mode="promise_in_bounds")
    return (knx != keys) | (lane == 15)


def _wid():
    return lax.axis_index("s") * NC + lax.axis_index("c")


def _sc_mesh():
    return plsc.VectorSubcoreMesh(core_axis_name="c", subcore_axis_name="s")


def _mm_body(f_ref, w_ref, b_ref, at_ref, pz_ref, h_ref, a_ref, h16_ref):
    h = jnp.dot(f_ref[...], w_ref[...], preferred_element_type=jnp.float32)
    h = h + b_ref[0:1, :]
    h_ref[...] = h
    a_ref[...] = jnp.dot(h, at_ref[...], preferred_element_type=jnp.float32)
    hsw = jnp.dot(h, pz_ref[...], preferred_element_type=jnp.float32)
    h16_ref[...] = hsw.astype(jnp.bfloat16)


def _mm_res_body(x_ref, p0_ref, p1_ref, w_ref, b_ref, at_ref, pz_ref,
                 f_ref, h_ref, a_ref, h16_ref):
    f = x_ref[...] + p0_ref[...] + p1_ref[...]
    f_ref[...] = f
    h = jnp.dot(f, w_ref[...], preferred_element_type=jnp.float32)
    h = h + b_ref[0:1, :]
    h_ref[...] = h
    a_ref[...] = jnp.dot(h, at_ref[...], preferred_element_type=jnp.float32)
    hsw = jnp.dot(h, pz_ref[...], preferred_element_type=jnp.float32)
    h16_ref[...] = hsw.astype(jnp.bfloat16)


_ROWS = 1000
_GRID = N // _ROWS


def _tc_project(f, W, b8, att2, pz):
    return pl.pallas_call(
        _mm_body,
        grid=(_GRID,),
        in_specs=[
            pl.BlockSpec((_ROWS, D), lambda i: (i, 0)),
            pl.BlockSpec((D, D), lambda i: (0, 0)),
            pl.BlockSpec((8, D), lambda i: (0, 0)),
            pl.BlockSpec((D, D), lambda i: (0, 0)),
            pl.BlockSpec((D, D), lambda i: (0, 0)),
        ],
        out_specs=[
            pl.BlockSpec((_ROWS, D), lambda i: (i, 0)),
            pl.BlockSpec((_ROWS, D), lambda i: (i, 0)),
            pl.BlockSpec((_ROWS, D), lambda i: (i, 0)),
        ],
        out_shape=[
            jax.ShapeDtypeStruct((N, D), jnp.float32),
            jax.ShapeDtypeStruct((N, D), jnp.float32),
            jax.ShapeDtypeStruct((N, D), jnp.bfloat16),
        ],
    )(f, W, b8, att2, pz)


def _tc_project_res(x, p0, p1, W, b8, att2, pz):
    return pl.pallas_call(
        _mm_res_body,
        grid=(_GRID,),
        in_specs=[
            pl.BlockSpec((_ROWS, D), lambda i: (i, 0)),
            pl.BlockSpec((_ROWS, D), lambda i: (i, 0)),
            pl.BlockSpec((_ROWS, D), lambda i: (i, 0)),
            pl.BlockSpec((D, D), lambda i: (0, 0)),
            pl.BlockSpec((8, D), lambda i: (0, 0)),
            pl.BlockSpec((D, D), lambda i: (0, 0)),
            pl.BlockSpec((D, D), lambda i: (0, 0)),
        ],
        out_specs=[
            pl.BlockSpec((_ROWS, D), lambda i: (i, 0)),
            pl.BlockSpec((_ROWS, D), lambda i: (i, 0)),
            pl.BlockSpec((_ROWS, D), lambda i: (i, 0)),
            pl.BlockSpec((_ROWS, D), lambda i: (i, 0)),
        ],
        out_shape=[
            jax.ShapeDtypeStruct((N, D), jnp.float32),
            jax.ShapeDtypeStruct((N, D), jnp.float32),
            jax.ShapeDtypeStruct((N, D), jnp.float32),
            jax.ShapeDtypeStruct((N, D), jnp.bfloat16),
        ],
    )(x, p0, p1, W, b8, att2, pz)


def _zm_body(ad_hbm, as_hbm, srcp, dstp, gp, z_out, m_parts,
             ad_t, as_t, src_t, dst_t, g_t, z_t, m_loc, mout, cmb, shm):
    c = lax.axis_index("c")
    s = lax.axis_index("s")
    wid = _wid()
    pltpu.sync_copy(ad_hbm, ad_t)
    pltpu.sync_copy(as_hbm, as_t)
    pltpu.sync_copy(srcp.at[wid], src_t)
    pltpu.sync_copy(dstp.at[wid], dst_t)
    pltpu.sync_copy(gp.at[wid], g_t)

    def init(i, _):
        m_loc[pl.ds(i * 16, 16)] = jnp.full((16,), _NEG, jnp.float32)
        return 0

    lax.fori_loop(0, NP // 16, init, 0)

    lane = lax.broadcasted_iota(jnp.int32, (16,), 0)

    def grp(r, _):
        for k in range(G8):
            sl = pl.ds(k * 16, 16)
            sv = src_t[r, sl]
            dv = dst_t[r, sl]
            gv = g_t[r, sl]
            av = plsc.load_gather(ad_t, [dv])
            bv = plsc.load_gather(as_t, [sv])
            t = av + bv
            lr = jnp.where(t >= 0.0, t, t * 0.2)
            z = (lr + gv) * (1.0 / TEMP)
            z_t[pl.ds(r * CH + k * 16, 16)] = z
            dk, zk = plsc.sort_key_val(dv, z)
            zc = _seg_prefix(jnp.maximum, zk, dk, lane)
            end = _run_end(dk, lane)
            mo = plsc.load_gather(m_loc, [dk])
            plsc.store_scatter(m_loc, [dk], jnp.maximum(mo, zc), mask=end)
        return 0

    lax.fori_loop(0, NCH, grp, 0)
    pltpu.sync_copy(z_t, z_out.at[wid])

    pltpu.sync_copy(m_loc, shm.at[s])
    plsc.subcore_barrier()
    pltpu.sync_copy(shm.at[:, pl.ds(s * NPS, NPS)], cmb)

    def red(i, _):
        sl = pl.ds(i * 16, 16)
        acc = cmb[0, sl]
        for t in range(1, NS):
            acc = jnp.maximum(acc, cmb[t, sl])
        mout[sl] = acc
        return 0

    lax.fori_loop(0, NPS // 16, red, 0)
    pltpu.sync_copy(mout, m_parts.at[c, pl.ds(s * NPS, NPS)])


def _sc_zm(ad, asrc, srcp, dstp, gp):
    return pl.kernel(
        _zm_body,
        out_type=[
            jax.ShapeDtypeStruct((NT, EPTP), jnp.float32),
            jax.ShapeDtypeStruct((NC, NP), jnp.float32),
        ],
        mesh=_sc_mesh(),
        compiler_params=pltpu.CompilerParams(needs_layout_passes=False),
        scratch_types=[
            pltpu.VMEM((NP,), jnp.float32),
            pltpu.VMEM((NP,), jnp.float32),
            pltpu.VMEM((NCH, CH), jnp.int32),
            pltpu.VMEM((NCH, CH), jnp.int32),
            pltpu.VMEM((NCH, CH), jnp.float32),
            pltpu.VMEM((EPTP,), jnp.float32),
            pltpu.VMEM((NP,), jnp.float32),
            pltpu.VMEM((NPS,), jnp.float32),
            pltpu.VMEM((NS, NPS), jnp.float32),
            pltpu.VMEM_SHARED((NS, NP), jnp.float32),
        ],
    )(ad, asrc, srcp, dstp, gp)


def _es_body(m_parts, z_in, dstp, s_parts,
             m_a, m_b, z_t, dst_t, s_loc, sout, cmb, shm):
    c = lax.axis_index("c")
    s = lax.axis_index("s")
    wid = _wid()
    pltpu.sync_copy(m_parts.at[0], m_a)
    pltpu.sync_copy(m_parts.at[1], m_b)
    pltpu.sync_copy(z_in.at[wid], z_t)
    pltpu.sync_copy(dstp.at[wid], dst_t)

    def init(i, _):
        sl = pl.ds(i * 16, 16)
        m_a[sl] = jnp.maximum(m_a[sl], m_b[sl])
        s_loc[sl] = jnp.zeros((16,), jnp.float32)
        return 0

    lax.fori_loop(0, NP // 16, init, 0)

    lane = lax.broadcasted_iota(jnp.int32, (16,), 0)

    def grp(r, _):
        for k in range(G8):
            sl = pl.ds(k * 16, 16)
            zv = z_t[pl.ds(r * CH + k * 16, 16)]
            dv = dst_t[r, sl]
            mv = plsc.load_gather(m_a, [dv])
            ev = jnp.exp(zv - mv)
            dk, ek = plsc.sort_key_val(dv, ev)
            ec = _seg_prefix(jnp.add, ek, dk, lane)
            end = _run_end(dk, lane)
            so = plsc.load_gather(s_loc, [dk])
            plsc.store_scatter(s_loc, [dk], so + ec, mask=end)
        return 0

    lax.fori_loop(0, NCH, grp, 0)

    pltpu.sync_copy(s_loc, shm.at[s])
    plsc.subcore_barrier()
    pltpu.sync_copy(shm.at[:, pl.ds(s * NPS, NPS)], cmb)

    def red(i, _):
        sl = pl.ds(i * 16, 16)
        acc = cmb[0, sl]
        for t in range(1, NS):
            acc = acc + cmb[t, sl]
        sout[sl] = acc
        return 0

    lax.fori_loop(0, NPS // 16, red, 0)
    pltpu.sync_copy(sout, s_parts.at[c, pl.ds(s * NPS, NPS)])


def _sc_es(m_parts, z_in, dstp):
    return pl.kernel(
        _es_body,
        out_type=jax.ShapeDtypeStruct((NC, NP), jnp.float32),
        mesh=_sc_mesh(),
        compiler_params=pltpu.CompilerParams(needs_layout_passes=False),
        scratch_types=[
            pltpu.VMEM((NP,), jnp.float32),
            pltpu.VMEM((NP,), jnp.float32),
            pltpu.VMEM((EPTP,), jnp.float32),
            pltpu.VMEM((NCH, CH), jnp.int32),
            pltpu.VMEM((NP,), jnp.float32),
            pltpu.VMEM((NPS,), jnp.float32),
            pltpu.VMEM((NS, NPS), jnp.float32),
            pltpu.VMEM_SHARED((NS, NP), jnp.float32),
        ],
    )(m_parts, z_in, dstp)


def _al_body(m_parts, s_parts, z_in, dstp, al_out,
             m_a, m_b, s_tmp, z_t, dst_t, al_t):
    wid = _wid()
    pltpu.sync_copy(m_parts.at[0], m_a)
    pltpu.sync_copy(m_parts.at[1], m_b)
    pltpu.sync_copy(z_in.at[wid], z_t)
    pltpu.sync_copy(dstp.at[wid], dst_t)

    def initm(i, _):
        sl = pl.ds(i * 16, 16)
        m_a[sl] = jnp.maximum(m_a[sl], m_b[sl])
        return 0

    lax.fori_loop(0, NP // 16, initm, 0)
    pltpu.sync_copy(s_parts.at[0], m_b)
    for q in range(NP // NPS):
        pltpu.sync_copy(s_parts.at[1, pl.ds(q * NPS, NPS)], s_tmp)

        def inits(i, _):
            sl = pl.ds(i * 16, 16)
            m_b[pl.ds(q * NPS + i * 16, 16)] = (
                m_b[pl.ds(q * NPS + i * 16, 16)] + s_tmp[sl])
            return 0

        lax.fori_loop(0, NPS // 16, inits, 0)

    lane = lax.broadcasted_iota(jnp.int32, (16,), 0)

    def grp(r, _):
        for k in range(G8):
            fl = pl.ds(r * CH + k * 16, 16)
            zv = z_t[fl]
            dv = dst_t[r, pl.ds(k * 16, 16)]
            mv = plsc.load_gather(m_a, [dv])
            sv = plsc.load_gather(m_b, [dv])
            av = jnp.exp(zv - mv) / (sv + 1e-16)
            pos = r * CH + k * 16 + lane
            av = jnp.where(pos < EPT, av, 0.0)
            al_t[fl] = av
        return 0

    lax.fori_loop(0, NCH, grp, 0)
    pltpu.sync_copy(al_t, al_out.at[wid])


def _sc_al(m_parts, s_parts, z_in, dstp):
    return pl.kernel(
        _al_body,
        out_type=jax.ShapeDtypeStruct((NT, EPTP), jnp.float32),
        mesh=_sc_mesh(),
        compiler_params=pltpu.CompilerParams(needs_layout_passes=False),
        scratch_types=[
            pltpu.VMEM((NP,), jnp.float32),
            pltpu.VMEM((NP,), jnp.float32),
            pltpu.VMEM((NPS,), jnp.float32),
            pltpu.VMEM((EPTP,), jnp.float32),
            pltpu.VMEM((NCH, CH), jnp.int32),
            pltpu.VMEM((EPTP,), jnp.float32),
        ],
    )(m_parts, s_parts, z_in, dstp)


def _agg_body(al_in, srcp, dstp, h16_hbm, out_parts,
              src_t, al_h, dst2_h, rows0, rows1, rowsf0, rowsf1,
              out_sh, sem0, sem1, ssem0, ssem1):
    c = lax.axis_index("c")
    s = lax.axis_index("s")
    wid = _wid()
    pltpu.sync_copy(srcp.at[wid], src_t)

    def zero_rows(r, _):
        for dcol in range(D // 16):
            rowsf0[r, pl.ds(dcol * 16, 16)] = jnp.zeros((16,), jnp.float32)
        return 0

    lax.fori_loop(0, CH, zero_rows, 0)
    for q in range(NPS // CH):
        pltpu.sync_copy(rowsf0, out_sh.at[pl.ds(s * NPS + q * CH, CH)])
    plsc.subcore_barrier()

    bufs = (rows0, rows1)
    sems = (sem0, sem1)
    stg = (rowsf0, rowsf1)
    ssems = (ssem0, ssem1)
    himask = jnp.full((16,), -65536, jnp.int32)

    def compute_chunk(rl, rows, rf):
        def grp(g, _):
            av = al_h[pl.ds(rl * CH + g * 16, 16)]
            for j in range(16):
                asp = av.at[jnp.full((16,), j, jnp.int32)].get(
                    mode="promise_in_bounds")
                for d2 in range(D // 32):
                    vi = rows[g * 16 + j, pl.ds(d2 * 16, 16)]
                    lo = plsc.bitcast(vi << 16, jnp.float32)
                    hi = plsc.bitcast(vi & himask, jnp.float32)
                    rf[g * 16 + j, pl.ds(d2 * 32, 16)] = lo * asp
                    rf[g * 16 + j, pl.ds(d2 * 32 + 16, 16)] = hi * asp
            return 0

        lax.fori_loop(0, G8, grp, 0)

    def gather_step(base, rl, p):
        nxt = jnp.minimum(rl + 1, NCHS - 1)
        pltpu.async_copy(
            h16_hbm.at[src_t.at[base + nxt]], bufs[1 - p], sems[1 - p])
        pltpu.make_async_copy(
            h16_hbm.at[src_t.at[base + rl]], bufs[p], sems[p]).wait()

    for piece in range(NCH // NCHS):
        base = piece * NCHS
        pltpu.sync_copy(al_in.at[wid, pl.ds(base * CH, NCHS * CH)], al_h)
        pltpu.sync_copy(dstp.at[wid, pl.ds(base, NCHS)], dst2_h.at[:, 0])
        pltpu.async_copy(h16_hbm.at[src_t.at[base]], bufs[0], sems[0])

        def pair(rr, _):
            for p in (0, 1):
                rl = rr * 2 + p
                gather_step(base, rl, p)

                @pl.when(rr > 0)
                def _():
                    pltpu.make_async_copy(
                        stg[p], out_sh.at[dst2_h.at[0, 0]], ssems[p]).wait()

                compute_chunk(rl, bufs[p], stg[p])
                pltpu.async_copy(stg[p], out_sh.at[dst2_h.at[rl, 0]],
                                 ssems[p], add=True)
            return 0

        lax.fori_loop(0, NCHS // 2, pair, 0)
        pltpu.make_async_copy(
            h16_hbm.at[src_t.at[base + NCHS - 1]], bufs[0], sems[0]).wait()
        for p in (0, 1):
            pltpu.make_async_copy(
                stg[p], out_sh.at[dst2_h.at[0, 0]], ssems[p]).wait()

    plsc.subcore_barrier()
    pltpu.sync_copy(out_sh.at[pl.ds(s * NPS, NPS)],
                    out_parts.at[c, pl.ds(s * NPS, NPS)])


def _sc_agg(al_in, srcp, dstp, h16):
    return pl.kernel(
        _agg_body,
        out_type=jax.ShapeDtypeStruct((NC, NP, D), jnp.float32),
        mesh=_sc_mesh(),
        compiler_params=pltpu.CompilerParams(
            needs_layout_passes=False, use_tc_tiling_on_sc=False),
        scratch_types=[
            pltpu.VMEM((NCH, CH), jnp.int32),
            pltpu.VMEM((NCHS * CH,), jnp.float32),
            pltpu.VMEM((NCHS, 2, CH), jnp.int32),
            pltpu.VMEM((CH, D // 2), jnp.int32),
            pltpu.VMEM((CH, D // 2), jnp.int32),
            pltpu.VMEM((CH, D), jnp.float32),
            pltpu.VMEM((CH, D), jnp.float32),
            pltpu.VMEM_SHARED((NP, D), jnp.float32),
            pltpu.SemaphoreType.DMA,
            pltpu.SemaphoreType.DMA,
            pltpu.SemaphoreType.DMA,
            pltpu.SemaphoreType.DMA,
        ],
    )(al_in, srcp, dstp, h16)


_RPT = ROOTS // NT


def _root_body(p0, p1, f1, roots, out, ridx, r0, r1, r2, o, sem):
    wid = _wid()
    sl = pl.ds(wid * _RPT, _RPT)
    pltpu.sync_copy(roots.at[sl], ridx)
    pltpu.async_copy(p0.at[ridx], r0, sem).wait()
    pltpu.async_copy(p1.at[ridx], r1, sem).wait()
    pltpu.async_copy(f1.at[ridx], r2, sem).wait()

    def add(i, _):
        for dcol in range(D // 16):
            dsl = pl.ds(dcol * 16, 16)
            o[i, dsl] = r0[i, dsl] + r1[i, dsl] + r2[i, dsl]
        return 0

    lax.fori_loop(0, _RPT, add, 0)
    pltpu.sync_copy(o, out.at[sl])


def _sc_root(p0, p1, f1, roots):
    return pl.kernel(
        _root_body,
        out_type=jax.ShapeDtypeStruct((ROOTS, D), jnp.float32),
        mesh=_sc_mesh(),
        compiler_params=pltpu.CompilerParams(needs_layout_passes=False),
        scratch_types=[
            pltpu.VMEM((_RPT,), jnp.int32),
            pltpu.VMEM((_RPT, D), jnp.float32),
            pltpu.VMEM((_RPT, D), jnp.float32),
            pltpu.VMEM((_RPT, D), jnp.float32),
            pltpu.VMEM((_RPT, D), jnp.float32),
            pltpu.SemaphoreType.DMA,
        ],
    )(p0, p1, f1, roots)


def _to_tiles(a, pad_val):
    a2 = a.reshape(NT, EPT)
    pad = jnp.full((NT, EPTP - EPT), pad_val, a2.dtype)
    return jnp.concatenate([a2, pad], axis=1).reshape(NT, NCH, CH)


def kernel(x, edge_index, root_index, W, b, att):
    key = jax.random.key(42)
    gp_hops = []
    for i in range(N_HOPS):
        u = jax.random.uniform(jax.random.fold_in(key, i), (E,),
                               minval=1e-6, maxval=1.0 - 1e-6)
        gp_hops.append(_to_tiles(-jnp.log(-jnp.log(u)), _NEG))

    srcp = _to_tiles(edge_index[0], 0)
    dstp = _to_tiles(edge_index[1], 0)
    b8 = jnp.broadcast_to(b.reshape(1, D), (8, D))
    att2 = jnp.zeros((D, D), jnp.float32)
    att2 = att2.at[:, 0].set(att[:D]).at[:, 1].set(att[D:])
    import numpy as _np
    perm_src = _np.zeros((D,), _np.int32)
    for d2 in range(D // 32):
        for j in range(16):
            perm_src[d2 * 32 + 2 * j] = d2 * 32 + j
            perm_src[d2 * 32 + 2 * j + 1] = d2 * 32 + 16 + j
    pz = jnp.zeros((D, D), jnp.float32)
    pz = pz.at[jnp.asarray(perm_src), jnp.arange(D)].set(1.0)

    def hop(h, a, gp, h16):
        ad = jnp.concatenate([a[:, 0], jnp.zeros((NP - N,), jnp.float32)])
        asrc = jnp.concatenate([a[:, 1], jnp.zeros((NP - N,), jnp.float32)])
        z, m_parts = _sc_zm(ad, asrc, srcp, dstp, gp)
        s_parts = _sc_es(m_parts, z, dstp)
        al = _sc_al(m_parts, s_parts, z, dstp)
        parts = _sc_agg(al, srcp, dstp, h16)
        return parts[0], parts[1]

    def as_i32(h16):
        return jax.lax.bitcast_convert_type(
            h16.reshape(N, D // 2, 2), jnp.int32)

    h1, a1, h16_1 = _tc_project(x, W, b8, att2, pz)
    p0, p1 = hop(h1, a1, gp_hops[0], as_i32(h16_1))
    f1, h2, a2, h16_2 = _tc_project_res(x, p0[:N], p1[:N], W, b8, att2, pz)
    q0, q1 = hop(h2, a2, gp_hops[1], as_i32(h16_2))
    return _sc_root(q0, q1, f1, root_index)


# speedup vs baseline: 9.5853x; 1.0026x over previous
import functools
import jax
import jax.numpy as jnp
from jax import lax
from jax.experimental import pallas as pl
from jax.experimental.pallas import tpu as pltpu
from jax.experimental.pallas import tpu_sc as plsc

N = 10000
E = 320000
D = 128
TEMP = 0.1
ROOTS = 1024
N_HOPS = 2

NC = 2
NS = 16
NT = NC * NS
EPT = E // NT
CH = 64
NCH = 160
EPTP = NCH * CH
NP = 10240
NPS = NP // NS
G8 = CH // 16
NCHS = NCH // 5

_NEG = -1e30


def _seg_prefix(op, vals, keys, lane):
    out = vals
    for sh in (1, 2, 4, 8):
        idx = jnp.maximum(lane - sh, 0)
        vs = out.at[idx].get(mode="promise_in_bounds")
        ks = keys.at[idx].get(mode="promise_in_bounds")
        take = (ks == keys) & (lane >= sh)
        out = jnp.where(take, op(out, vs), out)
    return out


def _run_end(keys, lane):
    nidx = jnp.minimum(lane + 1, 15)
    knx = keys.at[nidx].get(mode="promise_in_bounds")
    return (knx != keys) | (lane == 15)


def _wid():
    return lax.axis_index("s") * NC + lax.axis_index("c")


def _sc_mesh():
    return plsc.VectorSubcoreMesh(core_axis_name="c", subcore_axis_name="s")


def _mm_body(f_ref, w_ref, b_ref, at_ref, pz_ref, h_ref, a_ref, h16_ref):
    h = jnp.dot(f_ref[...], w_ref[...], preferred_element_type=jnp.float32)
    h = h + b_ref[0:1, :]
    h_ref[...] = h
    a_ref[...] = jnp.dot(h, at_ref[...], preferred_element_type=jnp.float32)
    hsw = jnp.dot(h, pz_ref[...], preferred_element_type=jnp.float32)
    h16_ref[...] = hsw.astype(jnp.bfloat16)


def _mm_res_body(x_ref, p0_ref, p1_ref, w_ref, b_ref, at_ref, pz_ref,
                 f_ref, h_ref, a_ref, h16_ref):
    f = x_ref[...] + p0_ref[...] + p1_ref[...]
    f_ref[...] = f
    h = jnp.dot(f, w_ref[...], preferred_element_type=jnp.float32)
    h = h + b_ref[0:1, :]
    h_ref[...] = h
    a_ref[...] = jnp.dot(h, at_ref[...], preferred_element_type=jnp.float32)
    hsw = jnp.dot(h, pz_ref[...], preferred_element_type=jnp.float32)
    h16_ref[...] = hsw.astype(jnp.bfloat16)


_ROWS = 1000
_GRID = N // _ROWS


def _tc_project(f, W, b8, att2, pz):
    return pl.pallas_call(
        _mm_body,
        grid=(_GRID,),
        in_specs=[
            pl.BlockSpec((_ROWS, D), lambda i: (i, 0)),
            pl.BlockSpec((D, D), lambda i: (0, 0)),
            pl.BlockSpec((8, D), lambda i: (0, 0)),
            pl.BlockSpec((D, D), lambda i: (0, 0)),
            pl.BlockSpec((D, D), lambda i: (0, 0)),
        ],
        out_specs=[
            pl.BlockSpec((_ROWS, D), lambda i: (i, 0)),
            pl.BlockSpec((_ROWS, D), lambda i: (i, 0)),
            pl.BlockSpec((_ROWS, D), lambda i: (i, 0)),
        ],
        out_shape=[
            jax.ShapeDtypeStruct((N, D), jnp.float32),
            jax.ShapeDtypeStruct((N, D), jnp.float32),
            jax.ShapeDtypeStruct((N, D), jnp.bfloat16),
        ],
    )(f, W, b8, att2, pz)


def _tc_project_res(x, p0, p1, W, b8, att2, pz):
    return pl.pallas_call(
        _mm_res_body,
        grid=(_GRID,),
        in_specs=[
            pl.BlockSpec((_ROWS, D), lambda i: (i, 0)),
            pl.BlockSpec((_ROWS, D), lambda i: (i, 0)),
            pl.BlockSpec((_ROWS, D), lambda i: (i, 0)),
            pl.BlockSpec((D, D), lambda i: (0, 0)),
            pl.BlockSpec((8, D), lambda i: (0, 0)),
            pl.BlockSpec((D, D), lambda i: (0, 0)),
            pl.BlockSpec((D, D), lambda i: (0, 0)),
        ],
        out_specs=[
            pl.BlockSpec((_ROWS, D), lambda i: (i, 0)),
            pl.BlockSpec((_ROWS, D), lambda i: (i, 0)),
            pl.BlockSpec((_ROWS, D), lambda i: (i, 0)),
            pl.BlockSpec((_ROWS, D), lambda i: (i, 0)),
        ],
        out_shape=[
            jax.ShapeDtypeStruct((N, D), jnp.float32),
            jax.ShapeDtypeStruct((N, D), jnp.float32),
            jax.ShapeDtypeStruct((N, D), jnp.float32),
            jax.ShapeDtypeStruct((N, D), jnp.bfloat16),
        ],
    )(x, p0, p1, W, b8, att2, pz)


def _zm_body(ad_hbm, as_hbm, srcp, dstp, gp, z_out, m_parts,
             ad_t, as_t, src_t, dst_t, g_t, z_t, m_loc, mout, cmb, shm):
    c = lax.axis_index("c")
    s = lax.axis_index("s")
    wid = _wid()
    pltpu.sync_copy(ad_hbm, ad_t)
    pltpu.sync_copy(as_hbm, as_t)
    pltpu.sync_copy(srcp.at[wid], src_t)
    pltpu.sync_copy(dstp.at[wid], dst_t)
    pltpu.sync_copy(gp.at[wid], g_t)

    def init(i, _):
        m_loc[pl.ds(i * 16, 16)] = jnp.full((16,), _NEG, jnp.float32)
        return 0

    lax.fori_loop(0, NP // 16, init, 0)

    lane = lax.broadcasted_iota(jnp.int32, (16,), 0)

    def grp(r, _):
        for k in range(G8):
            sl = pl.ds(k * 16, 16)
            sv = src_t[r, sl]
            dv = dst_t[r, sl]
            gv = g_t[r, sl]
            av = plsc.load_gather(ad_t, [dv])
            bv = plsc.load_gather(as_t, [sv])
            t = av + bv
            lr = jnp.where(t >= 0.0, t, t * 0.2)
            z = (lr + gv) * (1.0 / TEMP)
            z_t[pl.ds(r * CH + k * 16, 16)] = z
            dk, zk = plsc.sort_key_val(dv, z)
            zc = _seg_prefix(jnp.maximum, zk, dk, lane)
            end = _run_end(dk, lane)
            mo = plsc.load_gather(m_loc, [dk])
            plsc.store_scatter(m_loc, [dk], jnp.maximum(mo, zc), mask=end)
        return 0

    lax.fori_loop(0, NCH, grp, 0)
    pltpu.sync_copy(z_t, z_out.at[wid])

    pltpu.sync_copy(m_loc, shm.at[s])
    plsc.subcore_barrier()
    pltpu.sync_copy(shm.at[:, pl.ds(s * NPS, NPS)], cmb)

    def red(i, _):
        sl = pl.ds(i * 16, 16)
        acc = cmb[0, sl]
        for t in range(1, NS):
            acc = jnp.maximum(acc, cmb[t, sl])
        mout[sl] = acc
        return 0

    lax.fori_loop(0, NPS // 16, red, 0)
    pltpu.sync_copy(mout, m_parts.at[c, pl.ds(s * NPS, NPS)])


def _sc_zm(ad, asrc, srcp, dstp, gp):
    return pl.kernel(
        _zm_body,
        out_type=[
            jax.ShapeDtypeStruct((NT, EPTP), jnp.float32),
            jax.ShapeDtypeStruct((NC, NP), jnp.float32),
        ],
        mesh=_sc_mesh(),
        compiler_params=pltpu.CompilerParams(needs_layout_passes=False),
        scratch_types=[
            pltpu.VMEM((NP,), jnp.float32),
            pltpu.VMEM((NP,), jnp.float32),
            pltpu.VMEM((NCH, CH), jnp.int32),
            pltpu.VMEM((NCH, CH), jnp.int32),
            pltpu.VMEM((NCH, CH), jnp.float32),
            pltpu.VMEM((EPTP,), jnp.float32),
            pltpu.VMEM((NP,), jnp.float32),
            pltpu.VMEM((NPS,), jnp.float32),
            pltpu.VMEM((NS, NPS), jnp.float32),
            pltpu.VMEM_SHARED((NS, NP), jnp.float32),
        ],
    )(ad, asrc, srcp, dstp, gp)


def _esal_body(m_parts, z_in, dstp, al_out,
               m_a, m_b, z2, dst2, s_loc, sout, cmb, al_t, shm, sgl):
    c = lax.axis_index("c")
    s = lax.axis_index("s")
    pltpu.sync_copy(m_parts.at[0], m_a)
    pltpu.sync_copy(m_parts.at[1], m_b)
    pltpu.sync_copy(z_in.at[s * NC], z2.at[0])
    pltpu.sync_copy(z_in.at[s * NC + 1], z2.at[1])
    pltpu.sync_copy(dstp.at[s * NC], dst2.at[0])
    pltpu.sync_copy(dstp.at[s * NC + 1], dst2.at[1])

    def init(i, _):
        sl = pl.ds(i * 16, 16)
        m_a[sl] = jnp.maximum(m_a[sl], m_b[sl])
        s_loc[sl] = jnp.zeros((16,), jnp.float32)
        return 0

    lax.fori_loop(0, NP // 16, init, 0)

    lane = lax.broadcasted_iota(jnp.int32, (16,), 0)

    for u in range(NC):
        def grp(r, _, u=u):
            for k in range(G8):
                sl = pl.ds(k * 16, 16)
                zv = z2[u, pl.ds(r * CH + k * 16, 16)]
                dv = dst2[u, r, sl]
                mv = plsc.load_gather(m_a, [dv])
                ev = jnp.exp(zv - mv)
                dk, ek = plsc.sort_key_val(dv, ev)
                ec = _seg_prefix(jnp.add, ek, dk, lane)
                end = _run_end(dk, lane)
                so = plsc.load_gather(s_loc, [dk])
                plsc.store_scatter(s_loc, [dk], so + ec, mask=end)
            return 0

        lax.fori_loop(0, NCH, grp, 0)

    pltpu.sync_copy(s_loc, shm.at[s])
    plsc.subcore_barrier()
    pltpu.sync_copy(shm.at[:, pl.ds(s * NPS, NPS)], cmb)

    def red(i, _):
        sl = pl.ds(i * 16, 16)
        acc = cmb[0, sl]
        for t in range(1, NS):
            acc = acc + cmb[t, sl]
        sout[sl] = acc
        return 0

    lax.fori_loop(0, NPS // 16, red, 0)
    pltpu.sync_copy(sout, sgl.at[pl.ds(s * NPS, NPS)])
    plsc.subcore_barrier()
    pltpu.sync_copy(sgl, s_loc)

    def alp(r, _):
        for k in range(G8):
            fl = pl.ds(r * CH + k * 16, 16)
            zv = z2[c, fl]
            dv = dst2[c, r, pl.ds(k * 16, 16)]
            mv = plsc.load_gather(m_a, [dv])
            sv = plsc.load_gather(s_loc, [dv])
            av = jnp.exp(zv - mv) / (sv + 1e-16)
            pos = r * CH + k * 16 + lane
            av = jnp.where(pos < EPT, av, 0.0)
            al_t[fl] = av
        return 0

    lax.fori_loop(0, NCH, alp, 0)
    pltpu.sync_copy(al_t, al_out.at[s * NC + c])


def _sc_esal(m_parts, z_in, dstp):
    return pl.kernel(
        _esal_body,
        out_type=jax.ShapeDtypeStruct((NT, EPTP), jnp.float32),
        mesh=_sc_mesh(),
        compiler_params=pltpu.CompilerParams(needs_layout_passes=False),
        scratch_types=[
            pltpu.VMEM((NP,), jnp.float32),
            pltpu.VMEM((NP,), jnp.float32),
            pltpu.VMEM((NC, EPTP), jnp.float32),
            pltpu.VMEM((NC, NCH, CH), jnp.int32),
            pltpu.VMEM((NP,), jnp.float32),
            pltpu.VMEM((NPS,), jnp.float32),
            pltpu.VMEM((NS, NPS), jnp.float32),
            pltpu.VMEM((EPTP,), jnp.float32),
            pltpu.VMEM_SHARED((NS, NP), jnp.float32),
            pltpu.VMEM_SHARED((NP,), jnp.float32),
        ],
    )(m_parts, z_in, dstp)


def _agg_body(al_in, srcp, dstp, h16_hbm, out_parts,
              src_t, al_h, dst2_h, rows0, rows1, rowsf0, rowsf1,
              out_sh, sem0, sem1, ssem0, ssem1):
    c = lax.axis_index("c")
    s = lax.axis_index("s")
    wid = _wid()
    pltpu.sync_copy(srcp.at[wid], src_t)

    def zero_rows(r, _):
        for dcol in range(D // 16):
            rowsf0[r, pl.ds(dcol * 16, 16)] = jnp.zeros((16,), jnp.float32)
        return 0

    lax.fori_loop(0, CH, zero_rows, 0)
    for q in range(NPS // CH):
        pltpu.sync_copy(rowsf0, out_sh.at[pl.ds(s * NPS + q * CH, CH)])
    plsc.subcore_barrier()

    bufs = (rows0, rows1)
    sems = (sem0, sem1)
    stg = (rowsf0, rowsf1)
    ssems = (ssem0, ssem1)
    himask = jnp.full((16,), -65536, jnp.int32)

    def compute_chunk(rl, rows, rf):
        def grp(g, _):
            av = al_h[pl.ds(rl * CH + g * 16, 16)]
            for j in range(16):
                asp = av.at[jnp.full((16,), j, jnp.int32)].get(
                    mode="promise_in_bounds")
                for d2 in range(D // 32):
                    vi = rows[g * 16 + j, pl.ds(d2 * 16, 16)]
                    lo = plsc.bitcast(vi << 16, jnp.float32)
                    hi = plsc.bitcast(vi & himask, jnp.float32)
                    rf[g * 16 + j, pl.ds(d2 * 32, 16)] = lo * asp
                    rf[g * 16 + j, pl.ds(d2 * 32 + 16, 16)] = hi * asp
            return 0

        lax.fori_loop(0, G8, grp, 0)

    def gather_step(base, rl, p):
        nxt = jnp.minimum(rl + 1, NCHS - 1)
        pltpu.async_copy(
            h16_hbm.at[src_t.at[base + nxt]], bufs[1 - p], sems[1 - p])
        pltpu.make_async_copy(
            h16_hbm.at[src_t.at[base + rl]], bufs[p], sems[p]).wait()

    for piece in range(NCH // NCHS):
        base = piece * NCHS
        pltpu.sync_copy(al_in.at[wid, pl.ds(base * CH, NCHS * CH)], al_h)
        pltpu.sync_copy(dstp.at[wid, pl.ds(base, NCHS)], dst2_h.at[:, 0])
        pltpu.async_copy(h16_hbm.at[src_t.at[base]], bufs[0], sems[0])

        def pair(rr, _):
            for p in (0, 1):
                rl = rr * 2 + p
                gather_step(base, rl, p)

                @pl.when(rr > 0)
                def _():
                    pltpu.make_async_copy(
                        stg[p], out_sh.at[dst2_h.at[0, 0]], ssems[p]).wait()

                compute_chunk(rl, bufs[p], stg[p])
                pltpu.async_copy(stg[p], out_sh.at[dst2_h.at[rl, 0]],
                                 ssems[p], add=True)
            return 0

        lax.fori_loop(0, NCHS // 2, pair, 0)
        pltpu.make_async_copy(
            h16_hbm.at[src_t.at[base + NCHS - 1]], bufs[0], sems[0]).wait()
        for p in (0, 1):
            pltpu.make_async_copy(
                stg[p], out_sh.at[dst2_h.at[0, 0]], ssems[p]).wait()

    plsc.subcore_barrier()
    pltpu.sync_copy(out_sh.at[pl.ds(s * NPS, NPS)],
                    out_parts.at[c, pl.ds(s * NPS, NPS)])


def _sc_agg(al_in, srcp, dstp, h16):
    return pl.kernel(
        _agg_body,
        out_type=jax.ShapeDtypeStruct((NC, NP, D), jnp.float32),
        mesh=_sc_mesh(),
        compiler_params=pltpu.CompilerParams(
            needs_layout_passes=False, use_tc_tiling_on_sc=False),
        scratch_types=[
            pltpu.VMEM((NCH, CH), jnp.int32),
            pltpu.VMEM((NCHS * CH,), jnp.float32),
            pltpu.VMEM((NCHS, 2, CH), jnp.int32),
            pltpu.VMEM((CH, D // 2), jnp.int32),
            pltpu.VMEM((CH, D // 2), jnp.int32),
            pltpu.VMEM((CH, D), jnp.float32),
            pltpu.VMEM((CH, D), jnp.float32),
            pltpu.VMEM_SHARED((NP, D), jnp.float32),
            pltpu.SemaphoreType.DMA,
            pltpu.SemaphoreType.DMA,
            pltpu.SemaphoreType.DMA,
            pltpu.SemaphoreType.DMA,
        ],
    )(al_in, srcp, dstp, h16)


_RPT = ROOTS // NT


def _root_body(p0, p1, f1, roots, out, ridx, r0, r1, r2, o, sem):
    wid = _wid()
    sl = pl.ds(wid * _RPT, _RPT)
    pltpu.sync_copy(roots.at[sl], ridx)
    pltpu.async_copy(p0.at[ridx], r0, sem).wait()
    pltpu.async_copy(p1.at[ridx], r1, sem).wait()
    pltpu.async_copy(f1.at[ridx], r2, sem).wait()

    def add(i, _):
        for dcol in range(D // 16):
            dsl = pl.ds(dcol * 16, 16)
            o[i, dsl] = r0[i, dsl] + r1[i, dsl] + r2[i, dsl]
        return 0

    lax.fori_loop(0, _RPT, add, 0)
    pltpu.sync_copy(o, out.at[sl])


def _sc_root(p0, p1, f1, roots):
    return pl.kernel(
        _root_body,
        out_type=jax.ShapeDtypeStruct((ROOTS, D), jnp.float32),
        mesh=_sc_mesh(),
        compiler_params=pltpu.CompilerParams(needs_layout_passes=False),
        scratch_types=[
            pltpu.VMEM((_RPT,), jnp.int32),
            pltpu.VMEM((_RPT, D), jnp.float32),
            pltpu.VMEM((_RPT, D), jnp.float32),
            pltpu.VMEM((_RPT, D), jnp.float32),
            pltpu.VMEM((_RPT, D), jnp.float32),
            pltpu.SemaphoreType.DMA,
        ],
    )(p0, p1, f1, roots)


def _to_tiles(a, pad_val):
    a2 = a.reshape(NT, EPT)
    pad = jnp.full((NT, EPTP - EPT), pad_val, a2.dtype)
    return jnp.concatenate([a2, pad], axis=1).reshape(NT, NCH, CH)


def kernel(x, edge_index, root_index, W, b, att):
    key = jax.random.key(42)
    gp_hops = []
    for i in range(N_HOPS):
        u = jax.random.uniform(jax.random.fold_in(key, i), (E,),
                               minval=1e-6, maxval=1.0 - 1e-6)
        gp_hops.append(_to_tiles(-jnp.log(-jnp.log(u)), _NEG))

    srcp = _to_tiles(edge_index[0], 0)
    dstp = _to_tiles(edge_index[1], 0)
    b8 = jnp.broadcast_to(b.reshape(1, D), (8, D))
    att2 = jnp.zeros((D, D), jnp.float32)
    att2 = att2.at[:, 0].set(att[:D]).at[:, 1].set(att[D:])
    import numpy as _np
    perm_src = _np.zeros((D,), _np.int32)
    for d2 in range(D // 32):
        for j in range(16):
            perm_src[d2 * 32 + 2 * j] = d2 * 32 + j
            perm_src[d2 * 32 + 2 * j + 1] = d2 * 32 + 16 + j
    pz = jnp.zeros((D, D), jnp.float32)
    pz = pz.at[jnp.asarray(perm_src), jnp.arange(D)].set(1.0)

    def hop(h, a, gp, h16):
        ad = jnp.concatenate([a[:, 0], jnp.zeros((NP - N,), jnp.float32)])
        asrc = jnp.concatenate([a[:, 1], jnp.zeros((NP - N,), jnp.float32)])
        z, m_parts = _sc_zm(ad, asrc, srcp, dstp, gp)
        al = _sc_esal(m_parts, z, dstp)
        parts = _sc_agg(al, srcp, dstp, h16)
        return parts[0], parts[1]

    def as_i32(h16):
        return jax.lax.bitcast_convert_type(
            h16.reshape(N, D // 2, 2), jnp.int32)

    h1, a1, h16_1 = _tc_project(x, W, b8, att2, pz)
    p0, p1 = hop(h1, a1, gp_hops[0], as_i32(h16_1))
    f1, h2, a2, h16_2 = _tc_project_res(x, p0[:N], p1[:N], W, b8, att2, pz)
    q0, q1 = hop(h2, a2, gp_hops[1], as_i32(h16_2))
    return _sc_root(q0, q1, f1, root_index)
